# baseline (speedup 1.0000x reference)
import functools

import jax
import jax.numpy as jnp
from jax import lax
from jax.experimental import pallas as pl
from jax.experimental.pallas import tpu as pltpu
from jax.experimental.pallas import tpu_sc as plsc

F32 = jnp.float32
BF16 = jnp.bfloat16
I32 = jnp.int32

T = 4096
H = 1024
E = 8
CAP = 512
B = 2
S = 2048
DISP_ROWS = 4608
TRASH = 4096
NW = 32
RPW = T // NW
GW = 32
NBUF = 3
NCH = RPW // GW
INV_ROWS = 4104
IW = 128

_BLK1 = 512
_BLKR = 256
_BLKF = 256


def _stage1_body(x_ref, w_ref, b_ref, wg_ref, hid_ref, log_ref):
    xb = x_ref[...].astype(BF16)
    h = jnp.dot(xb, w_ref[...].astype(BF16), preferred_element_type=F32)
    h = h + b_ref[...]
    hid_ref[...] = h
    log_ref[...] = jnp.dot(h.astype(BF16), wg_ref[...].astype(BF16),
                           preferred_element_type=F32)


def _routing_body(l_ref, ss_ref, g_ref, cnt_ref, carry_ref):
    pid = pl.program_id(0)

    @pl.when(pid == 0)
    def _():
        carry_ref[...] = jnp.zeros_like(carry_ref)

    l = l_ref[...]
    n = l.shape[0]
    m = jnp.max(l, axis=1, keepdims=True)
    s = jnp.sum(jnp.exp(l - m), axis=1, keepdims=True)
    gv = 1.0 / s
    ei = lax.broadcasted_iota(I32, (n, E), 1)
    idx = jnp.min(jnp.where(l == m, ei, E), axis=1, keepdims=True)
    mask = (ei == idx).astype(F32)
    ri = lax.broadcasted_iota(I32, (n, n), 0)
    ci = lax.broadcasted_iota(I32, (n, n), 1)
    tril = (ci <= ri).astype(BF16)
    incl = jnp.dot(tril, mask.astype(BF16), preferred_element_type=F32)
    incl = jnp.sum(incl * mask, axis=1, keepdims=True)
    carry = carry_ref[...]
    base = jnp.sum(carry * mask, axis=1, keepdims=True)
    carry_ref[...] = carry + jnp.sum(mask, axis=0, keepdims=True)
    loc = base + incl - 1.0
    keep = loc < CAP
    locc = jnp.minimum(loc, CAP - 1.0).astype(I32)
    slot = idx * CAP + locc
    ss_ref[...] = jnp.where(keep, slot, TRASH)
    g_ref[...] = jnp.where(keep, gv, 0.0)
    cnt = carry_ref[...].astype(I32)
    cnt_ref[...] = jnp.concatenate(
        [cnt, jnp.zeros((1, 16 - E), I32)], axis=1)


def _ffn_body(d_ref, wa_ref, ba_ref, wb_ref, bb_ref, h_ref):
    lhs = d_ref[...].astype(BF16)
    t = jnp.dot(lhs, wa_ref[0].astype(BF16), preferred_element_type=F32)
    t = t + ba_ref[0]
    h = jnp.dot(t.astype(BF16), wb_ref[0].astype(BF16),
                preferred_element_type=F32)
    h = h + bb_ref[0]
    h_ref[...] = h


def _w2_body(r_ref, g_ref, w_ref, b_ref, wg_ref, o_ref, log_ref):
    g = g_ref[...]
    lhs = jnp.where(g > 0, r_ref[...] * g, 0.0).astype(BF16)
    o = jnp.dot(lhs, w_ref[...].astype(BF16), preferred_element_type=F32)
    o = o + b_ref[...]
    o_ref[...] = o
    log_ref[...] = jnp.dot(o.astype(BF16), wg_ref[...].astype(BF16),
                           preferred_element_type=F32)


def _final_body(h_ref, r_ref, g_ref, w3_ref, b3_ref, y_ref, out_ref,
                acch_ref, acco_ref):
    pid = pl.program_id(0)

    @pl.when(pid == 0)
    def _():
        acch_ref[...] = jnp.zeros_like(acch_ref)
        acco_ref[...] = jnp.zeros_like(acco_ref)
        out_ref[...] = jnp.zeros_like(out_ref)

    b = pid // (S // _BLKF)
    rowsel = (lax.broadcasted_iota(I32, (B, 1), 0) == b).astype(F32)
    hsum = jnp.sum(h_ref[...], axis=0, keepdims=True)
    g = g_ref[...]
    osum = jnp.sum(jnp.where(g > 0, r_ref[...] * g, 0.0),
                   axis=0, keepdims=True)
    acch_ref[...] += rowsel * hsum
    acco_ref[...] += rowsel * osum

    @pl.when(pid == pl.num_programs(0) - 1)
    def _():
        sent = acch_ref[...] * (1.0 / S)
        sent = sent + jnp.dot((acco_ref[...] * (1.0 / S)).astype(BF16),
                              w3_ref[...].astype(BF16),
                              preferred_element_type=F32)
        sent = sent + b3_ref[...]
        m = jnp.max(sent, axis=1, keepdims=True)
        lz = jnp.log(jnp.sum(jnp.exp(sent - m), axis=1, keepdims=True)) + m
        ci = lax.broadcasted_iota(I32, (B, H), 1)
        picked = jnp.sum(jnp.where(ci == y_ref[...], sent, 0.0),
                         axis=1, keepdims=True)
        out_ref[...] = jnp.sum(lz - picked, axis=0, keepdims=True) / B


def _stage1(xf, W1, b1, Wg1):
    grid = (T // _BLK1,)
    return pl.pallas_call(
        _stage1_body,
        grid=grid,
        in_specs=[
            pl.BlockSpec((_BLK1, H), lambda i: (i, 0)),
            pl.BlockSpec((H, H), lambda i: (0, 0)),
            pl.BlockSpec((1, H), lambda i: (0, 0)),
            pl.BlockSpec((H, E), lambda i: (0, 0)),
        ],
        out_specs=[
            pl.BlockSpec((_BLK1, H), lambda i: (i, 0)),
            pl.BlockSpec((_BLK1, E), lambda i: (i, 0)),
        ],
        out_shape=[
            jax.ShapeDtypeStruct((T, H), F32),
            jax.ShapeDtypeStruct((T, E), F32),
        ],
        compiler_params=pltpu.CompilerParams(
            dimension_semantics=("arbitrary",)),
    )(xf, W1, b1, Wg1)


def _routing(logits):
    grid = (T // _BLKR,)
    return pl.pallas_call(
        _routing_body,
        grid=grid,
        in_specs=[pl.BlockSpec((_BLKR, E), lambda i: (i, 0))],
        out_specs=[
            pl.BlockSpec((_BLKR, 1), lambda i: (i, 0)),
            pl.BlockSpec((_BLKR, 1), lambda i: (i, 0)),
            pl.BlockSpec((1, 16), lambda i: (0, 0)),
        ],
        out_shape=[
            jax.ShapeDtypeStruct((T, 1), I32),
            jax.ShapeDtypeStruct((T, 1), F32),
            jax.ShapeDtypeStruct((1, 16), I32),
        ],
        scratch_shapes=[pltpu.VMEM((1, E), F32)],
        compiler_params=pltpu.CompilerParams(
            dimension_semantics=("arbitrary",)),
    )(logits)


def _ffn(disp, Wa, ba, Wb, bb):
    grid = (E,)
    return pl.pallas_call(
        _ffn_body,
        grid=grid,
        in_specs=[
            pl.BlockSpec((CAP, H), lambda e: (e, 0)),
            pl.BlockSpec((1, H, H), lambda e: (e, 0, 0)),
            pl.BlockSpec((1, 1, H), lambda e: (e, 0, 0)),
            pl.BlockSpec((1, H, H), lambda e: (e, 0, 0)),
            pl.BlockSpec((1, 1, H), lambda e: (e, 0, 0)),
        ],
        out_specs=[pl.BlockSpec((CAP, H), lambda e: (e, 0))],
        out_shape=[jax.ShapeDtypeStruct((T, H), F32)],
        compiler_params=pltpu.CompilerParams(
            dimension_semantics=("arbitrary",)),
    )(disp, Wa, ba, Wb, bb)[0]


def _w2(rows, gate, W2, b2, Wg2):
    grid = (T // _BLK1,)
    return pl.pallas_call(
        _w2_body,
        grid=grid,
        in_specs=[
            pl.BlockSpec((_BLK1, H), lambda i: (i, 0)),
            pl.BlockSpec((_BLK1, 1), lambda i: (i, 0)),
            pl.BlockSpec((H, H), lambda i: (0, 0)),
            pl.BlockSpec((1, H), lambda i: (0, 0)),
            pl.BlockSpec((H, E), lambda i: (0, 0)),
        ],
        out_specs=[
            pl.BlockSpec((_BLK1, H), lambda i: (i, 0)),
            pl.BlockSpec((_BLK1, E), lambda i: (i, 0)),
        ],
        out_shape=[
            jax.ShapeDtypeStruct((T, H), F32),
            jax.ShapeDtypeStruct((T, E), F32),
        ],
        compiler_params=pltpu.CompilerParams(
            dimension_semantics=("arbitrary",)),
    )(rows, gate, W2, b2, Wg2)


def _final(hidden, rows2, gate2, W3, b3, y2):
    grid = (T // _BLKF,)
    return pl.pallas_call(
        _final_body,
        grid=grid,
        in_specs=[
            pl.BlockSpec((_BLKF, H), lambda i: (i, 0)),
            pl.BlockSpec((_BLKF, H), lambda i: (i, 0)),
            pl.BlockSpec((_BLKF, 1), lambda i: (i, 0)),
            pl.BlockSpec((H, H), lambda i: (0, 0)),
            pl.BlockSpec((1, H), lambda i: (0, 0)),
            pl.BlockSpec((B, 1), lambda i: (0, 0)),
        ],
        out_specs=[pl.BlockSpec((1, 1), lambda i: (0, 0))],
        out_shape=[jax.ShapeDtypeStruct((1, 1), F32)],
        scratch_shapes=[pltpu.VMEM((B, H), F32), pltpu.VMEM((B, H), F32)],
        compiler_params=pltpu.CompilerParams(
            dimension_semantics=("arbitrary",)),
    )(hidden, rows2, gate2, W3, b3, y2)[0]


def _sc_mesh():
    return plsc.VectorSubcoreMesh(core_axis_name="c", subcore_axis_name="s")


def _sc_dispatch(src, slots2, slots1):
    scratch = [
        pltpu.VMEM((NCH, GW), I32),
        pltpu.VMEM((RPW,), I32),
        pltpu.VMEM((NBUF, GW, H), F32),
        pltpu.VMEM((RPW, IW), I32),
    ] + [pltpu.SemaphoreType.DMA] * (2 * NBUF + 1)

    @functools.partial(
        pl.kernel,
        out_type=[jax.ShapeDtypeStruct((DISP_ROWS, H), F32),
                  jax.ShapeDtypeStruct((INV_ROWS, IW), I32)],
        mesh=_sc_mesh(),
        scratch_types=scratch,
        compiler_params=pltpu.CompilerParams(needs_layout_passes=False),
    )
    def k(src_hbm, slot2_hbm, slot1_hbm, out_hbm, inv_hbm,
          idx_v, idxf_v, buf, tok_v, *sems):
        sin, sout, stok = sems[:NBUF], sems[NBUF:2 * NBUF], sems[2 * NBUF]
        wid = lax.axis_index("s") * 2 + lax.axis_index("c")
        base = wid * RPW
        pltpu.sync_copy(slot2_hbm.at[pl.ds(wid * NCH, NCH)], idx_v)
        pltpu.sync_copy(slot1_hbm.at[pl.ds(base, RPW)], idxf_v)
        for j in range(RPW):
            tok_v[j, pl.ds(0, 16)] = jnp.full((16,), base + j, I32)
        tok_cp = pltpu.async_copy(tok_v, inv_hbm.at[idxf_v], stok)

        ins = [None] * NCH
        outs = [None] * NCH
        for j in range(min(NBUF, NCH)):
            ins[j] = pltpu.async_copy(
                src_hbm.at[pl.ds(base + j * GW, GW)], buf.at[j % NBUF],
                sin[j % NBUF])
        for j in range(NCH):
            ins[j].wait()
            outs[j] = pltpu.async_copy(
                buf.at[j % NBUF], out_hbm.at[idx_v.at[j]], sout[j % NBUF])
            nxt = j + NBUF
            if nxt < NCH:
                outs[j].wait()
                ins[nxt] = pltpu.async_copy(
                    src_hbm.at[pl.ds(base + nxt * GW, GW)],
                    buf.at[nxt % NBUF], sin[nxt % NBUF])
        for j in range(max(0, NCH - NBUF), NCH):
            outs[j].wait()
        tok_cp.wait()

    return k(src, slots2, slots1)


def _sc_combine(h, inv16, counts):
    scratch = [
        pltpu.VMEM((RPW, IW), I32),
        pltpu.VMEM((NCH, GW), I32),
        pltpu.VMEM((NBUF, GW, H), F32),
        pltpu.VMEM((16,), I32),
    ] + [pltpu.SemaphoreType.DMA] * (2 * NBUF)

    @functools.partial(
        pl.kernel,
        out_type=jax.ShapeDtypeStruct((DISP_ROWS, H), F32),
        mesh=_sc_mesh(),
        scratch_types=scratch,
        compiler_params=pltpu.CompilerParams(needs_layout_passes=False),
    )
    def k(h_hbm, inv_hbm, cnt_hbm, out_hbm, inv_v, idx_v, buf, cnt_v, *sems):
        sin, sout = sems[:NBUF], sems[NBUF:]
        wid = lax.axis_index("s") * 2 + lax.axis_index("c")
        base = wid * RPW
        pltpu.sync_copy(inv_hbm.at[pl.ds(base, RPW)], inv_v)
        pltpu.sync_copy(cnt_hbm, cnt_v)
        e = base // CAP
        ce = plsc.load_gather(cnt_v, [jnp.full((16,), e, I32)])
        bound = e * CAP + jnp.minimum(ce, CAP)
        zeros = jnp.zeros((16,), I32)
        lane = lax.iota(I32, 16)
        for kk in range(RPW // 16):
            ids = plsc.load_gather(inv_v, [kk * 16 + lane, zeros])
            valid = (base + kk * 16 + lane) < bound
            vals = jnp.where(valid, ids, TRASH)
            idx_v[(kk * 16) // GW, pl.ds((kk * 16) % GW, 16)] = vals

        ins = [None] * NCH
        outs = [None] * NCH
        for j in range(min(NBUF, NCH)):
            ins[j] = pltpu.async_copy(
                h_hbm.at[pl.ds(base + j * GW, GW)], buf.at[j % NBUF],
                sin[j % NBUF])
        for j in range(NCH):
            ins[j].wait()
            outs[j] = pltpu.async_copy(
                buf.at[j % NBUF], out_hbm.at[idx_v.at[j]], sout[j % NBUF])
            nxt = j + NBUF
            if nxt < NCH:
                outs[j].wait()
                ins[nxt] = pltpu.async_copy(
                    h_hbm.at[pl.ds(base + nxt * GW, GW)],
                    buf.at[nxt % NBUF], sin[nxt % NBUF])
        for j in range(max(0, NCH - NBUF), NCH):
            outs[j].wait()

    return k(h, inv16, counts)


def kernel(x, y, W1, b1, Wg1, We1a, be1a, We1b, be1b, W2, b2, Wg2,
           We2a, be2a, We2b, be2b, W3, b3):
    xf = x.reshape(T, H)
    hidden16, logits1 = _stage1(xf, W1, b1.reshape(1, H), Wg1)

    ss1, gate1, cnt1 = _routing(logits1)
    disp1, inv1 = _sc_dispatch(hidden16, ss1.reshape(-1, GW), ss1.reshape(T))
    h1 = _ffn(disp1, We1a, be1a.reshape(E, 1, H),
              We1b, be1b.reshape(E, 1, H))
    rows1 = _sc_combine(h1, inv1, cnt1.reshape(16))

    out16, logits2 = _w2(rows1, gate1, W2, b2.reshape(1, H), Wg2)

    ss2, gate2, cnt2 = _routing(logits2)
    disp2, inv2 = _sc_dispatch(out16, ss2.reshape(-1, GW), ss2.reshape(T))
    h2 = _ffn(disp2, We2a, be2a.reshape(E, 1, H),
              We2b, be2b.reshape(E, 1, H))
    rows2 = _sc_combine(h2, inv2, cnt2.reshape(16))

    loss = _final(hidden16, rows2, gate2, W3,
                  b3.reshape(1, H), y.reshape(B, 1).astype(I32))
    return loss.reshape(())


# speedup vs baseline: 2.0010x; 2.0010x over previous
import functools

import jax
import jax.numpy as jnp
from jax import lax
from jax.experimental import pallas as pl
from jax.experimental.pallas import tpu as pltpu
from jax.experimental.pallas import tpu_sc as plsc

F32 = jnp.float32
BF16 = jnp.bfloat16
I32 = jnp.int32

T = 4096
H = 1024
E = 8
CAP = 512
B = 2
S = 2048
DISP_ROWS = 4608
TRASH = 4096
NW = 32
RPW = T // NW
GW = 32
NBUF = 3
NCH = RPW // GW
INV_ROWS = 4608
IW = 128

_BLK1 = 512
_BLKR = 256
_BLKF = 256


def _stage1_body(x_ref, w_ref, b_ref, wg_ref, hid_ref, log_ref):
    xb = x_ref[...].astype(BF16)
    h = jnp.dot(xb, w_ref[...].astype(BF16), preferred_element_type=F32)
    h = h + b_ref[...]
    hid_ref[...] = h
    log_ref[...] = jnp.dot(h.astype(BF16), wg_ref[...].astype(BF16),
                           preferred_element_type=F32)


def _routing_body(l_ref, ss_ref, g_ref, cnt_ref, carry_ref):
    pid = pl.program_id(0)

    @pl.when(pid == 0)
    def _():
        carry_ref[...] = jnp.zeros_like(carry_ref)

    l = l_ref[...]
    n = l.shape[0]
    m = jnp.max(l, axis=1, keepdims=True)
    s = jnp.sum(jnp.exp(l - m), axis=1, keepdims=True)
    gv = 1.0 / s
    ei = lax.broadcasted_iota(I32, (n, E), 1)
    idx = jnp.min(jnp.where(l == m, ei, E), axis=1, keepdims=True)
    mask = (ei == idx).astype(F32)
    ri = lax.broadcasted_iota(I32, (n, n), 0)
    ci = lax.broadcasted_iota(I32, (n, n), 1)
    tril = (ci <= ri).astype(BF16)
    incl = jnp.dot(tril, mask.astype(BF16), preferred_element_type=F32)
    incl = jnp.sum(incl * mask, axis=1, keepdims=True)
    carry = carry_ref[...]
    base = jnp.sum(carry * mask, axis=1, keepdims=True)
    carry_ref[...] = carry + jnp.sum(mask, axis=0, keepdims=True)
    loc = base + incl - 1.0
    keep = loc < CAP
    locc = jnp.minimum(loc, CAP - 1.0).astype(I32)
    slot = idx * CAP + locc
    tok = (lax.broadcasted_iota(I32, (n, 1), 0)
           + pid * n)
    trash = TRASH + (tok & (DISP_ROWS - TRASH - 1))
    ss_ref[...] = jnp.where(keep, slot, trash)
    g_ref[...] = jnp.where(keep, gv, 0.0)
    cnt = carry_ref[...].astype(I32)
    cnt_ref[...] = jnp.concatenate(
        [cnt, jnp.zeros((1, 16 - E), I32)], axis=1)


def _ffn_body(d_ref, wa_ref, ba_ref, wb_ref, bb_ref, h_ref):
    lhs = d_ref[...].astype(BF16)
    t = jnp.dot(lhs, wa_ref[0].astype(BF16), preferred_element_type=F32)
    t = t + ba_ref[0]
    h = jnp.dot(t.astype(BF16), wb_ref[0].astype(BF16),
                preferred_element_type=F32)
    h = h + bb_ref[0]
    h_ref[...] = h


def _w2_body(r_ref, g_ref, w_ref, b_ref, wg_ref, o_ref, log_ref):
    g = g_ref[...]
    lhs = jnp.where(g > 0, r_ref[...] * g, 0.0).astype(BF16)
    o = jnp.dot(lhs, w_ref[...].astype(BF16), preferred_element_type=F32)
    o = o + b_ref[...]
    o_ref[...] = o
    log_ref[...] = jnp.dot(o.astype(BF16), wg_ref[...].astype(BF16),
                           preferred_element_type=F32)


def _final_body(h_ref, r_ref, g_ref, w3_ref, b3_ref, y_ref, out_ref,
                acch_ref, acco_ref):
    pid = pl.program_id(0)

    @pl.when(pid == 0)
    def _():
        acch_ref[...] = jnp.zeros_like(acch_ref)
        acco_ref[...] = jnp.zeros_like(acco_ref)
        out_ref[...] = jnp.zeros_like(out_ref)

    b = pid // (S // _BLKF)
    rowsel = (lax.broadcasted_iota(I32, (B, 1), 0) == b).astype(F32)
    hsum = jnp.sum(h_ref[...], axis=0, keepdims=True)
    g = g_ref[...]
    osum = jnp.sum(jnp.where(g > 0, r_ref[...] * g, 0.0),
                   axis=0, keepdims=True)
    acch_ref[...] += rowsel * hsum
    acco_ref[...] += rowsel * osum

    @pl.when(pid == pl.num_programs(0) - 1)
    def _():
        sent = acch_ref[...] * (1.0 / S)
        sent = sent + jnp.dot((acco_ref[...] * (1.0 / S)).astype(BF16),
                              w3_ref[...].astype(BF16),
                              preferred_element_type=F32)
        sent = sent + b3_ref[...]
        m = jnp.max(sent, axis=1, keepdims=True)
        lz = jnp.log(jnp.sum(jnp.exp(sent - m), axis=1, keepdims=True)) + m
        ci = lax.broadcasted_iota(I32, (B, H), 1)
        picked = jnp.sum(jnp.where(ci == y_ref[...], sent, 0.0),
                         axis=1, keepdims=True)
        out_ref[...] = jnp.sum(lz - picked, axis=0, keepdims=True) / B


def _stage1(xf, W1, b1, Wg1):
    grid = (T // _BLK1,)
    return pl.pallas_call(
        _stage1_body,
        grid=grid,
        in_specs=[
            pl.BlockSpec((_BLK1, H), lambda i: (i, 0)),
            pl.BlockSpec((H, H), lambda i: (0, 0)),
            pl.BlockSpec((1, H), lambda i: (0, 0)),
            pl.BlockSpec((H, E), lambda i: (0, 0)),
        ],
        out_specs=[
            pl.BlockSpec((_BLK1, H), lambda i: (i, 0)),
            pl.BlockSpec((_BLK1, E), lambda i: (i, 0)),
        ],
        out_shape=[
            jax.ShapeDtypeStruct((T, H), F32),
            jax.ShapeDtypeStruct((T, E), F32),
        ],
        compiler_params=pltpu.CompilerParams(
            dimension_semantics=("arbitrary",)),
    )(xf, W1, b1, Wg1)


def _routing(logits):
    grid = (T // _BLKR,)
    return pl.pallas_call(
        _routing_body,
        grid=grid,
        in_specs=[pl.BlockSpec((_BLKR, E), lambda i: (i, 0))],
        out_specs=[
            pl.BlockSpec((_BLKR, 1), lambda i: (i, 0)),
            pl.BlockSpec((_BLKR, 1), lambda i: (i, 0)),
            pl.BlockSpec((1, 16), lambda i: (0, 0)),
        ],
        out_shape=[
            jax.ShapeDtypeStruct((T, 1), I32),
            jax.ShapeDtypeStruct((T, 1), F32),
            jax.ShapeDtypeStruct((1, 16), I32),
        ],
        scratch_shapes=[pltpu.VMEM((1, E), F32)],
        compiler_params=pltpu.CompilerParams(
            dimension_semantics=("arbitrary",)),
    )(logits)


def _ffn(disp, Wa, ba, Wb, bb):
    grid = (E,)
    return pl.pallas_call(
        _ffn_body,
        grid=grid,
        in_specs=[
            pl.BlockSpec((CAP, H), lambda e: (e, 0)),
            pl.BlockSpec((1, H, H), lambda e: (e, 0, 0)),
            pl.BlockSpec((1, 1, H), lambda e: (e, 0, 0)),
            pl.BlockSpec((1, H, H), lambda e: (e, 0, 0)),
            pl.BlockSpec((1, 1, H), lambda e: (e, 0, 0)),
        ],
        out_specs=[pl.BlockSpec((CAP, H), lambda e: (e, 0))],
        out_shape=[jax.ShapeDtypeStruct((T, H), F32)],
        compiler_params=pltpu.CompilerParams(
            dimension_semantics=("arbitrary",)),
    )(disp, Wa, ba, Wb, bb)[0]


def _w2(rows, gate, W2, b2, Wg2):
    grid = (T // _BLK1,)
    return pl.pallas_call(
        _w2_body,
        grid=grid,
        in_specs=[
            pl.BlockSpec((_BLK1, H), lambda i: (i, 0)),
            pl.BlockSpec((_BLK1, 1), lambda i: (i, 0)),
            pl.BlockSpec((H, H), lambda i: (0, 0)),
            pl.BlockSpec((1, H), lambda i: (0, 0)),
            pl.BlockSpec((H, E), lambda i: (0, 0)),
        ],
        out_specs=[
            pl.BlockSpec((_BLK1, H), lambda i: (i, 0)),
            pl.BlockSpec((_BLK1, E), lambda i: (i, 0)),
        ],
        out_shape=[
            jax.ShapeDtypeStruct((T, H), F32),
            jax.ShapeDtypeStruct((T, E), F32),
        ],
        compiler_params=pltpu.CompilerParams(
            dimension_semantics=("arbitrary",)),
    )(rows, gate, W2, b2, Wg2)


def _final(hidden, rows2, gate2, W3, b3, y2):
    grid = (T // _BLKF,)
    return pl.pallas_call(
        _final_body,
        grid=grid,
        in_specs=[
            pl.BlockSpec((_BLKF, H), lambda i: (i, 0)),
            pl.BlockSpec((_BLKF, H), lambda i: (i, 0)),
            pl.BlockSpec((_BLKF, 1), lambda i: (i, 0)),
            pl.BlockSpec((H, H), lambda i: (0, 0)),
            pl.BlockSpec((1, H), lambda i: (0, 0)),
            pl.BlockSpec((B, 1), lambda i: (0, 0)),
        ],
        out_specs=[pl.BlockSpec((1, 1), lambda i: (0, 0))],
        out_shape=[jax.ShapeDtypeStruct((1, 1), F32)],
        scratch_shapes=[pltpu.VMEM((B, H), F32), pltpu.VMEM((B, H), F32)],
        compiler_params=pltpu.CompilerParams(
            dimension_semantics=("arbitrary",)),
    )(hidden, rows2, gate2, W3, b3, y2)[0]


def _sc_mesh():
    return plsc.VectorSubcoreMesh(core_axis_name="c", subcore_axis_name="s")


def _sc_dispatch(src, slots2, slots1):
    scratch = [
        pltpu.VMEM((NCH, GW), I32),
        pltpu.VMEM((RPW,), I32),
        pltpu.VMEM((NBUF, GW, H), F32),
        pltpu.VMEM((RPW, IW), I32),
    ] + [pltpu.SemaphoreType.DMA] * (2 * NBUF + 1)

    @functools.partial(
        pl.kernel,
        out_type=[jax.ShapeDtypeStruct((DISP_ROWS, H), F32),
                  jax.ShapeDtypeStruct((INV_ROWS, IW), I32)],
        mesh=_sc_mesh(),
        scratch_types=scratch,
        compiler_params=pltpu.CompilerParams(needs_layout_passes=False),
    )
    def k(src_hbm, slot2_hbm, slot1_hbm, out_hbm, inv_hbm,
          idx_v, idxf_v, buf, tok_v, *sems):
        sin, sout, stok = sems[:NBUF], sems[NBUF:2 * NBUF], sems[2 * NBUF]
        wid = lax.axis_index("s") * 2 + lax.axis_index("c")
        base = wid * RPW
        pltpu.sync_copy(slot2_hbm.at[pl.ds(wid * NCH, NCH)], idx_v)
        pltpu.sync_copy(slot1_hbm.at[pl.ds(base, RPW)], idxf_v)
        for j in range(RPW):
            tok_v[j, pl.ds(0, 16)] = jnp.full((16,), base + j, I32)
        tok_cp = pltpu.async_copy(tok_v, inv_hbm.at[idxf_v], stok)

        ins = [None] * NCH
        outs = [None] * NCH
        for j in range(min(NBUF, NCH)):
            ins[j] = pltpu.async_copy(
                src_hbm.at[pl.ds(base + j * GW, GW)], buf.at[j % NBUF],
                sin[j % NBUF])
        for j in range(NCH):
            ins[j].wait()
            outs[j] = pltpu.async_copy(
                buf.at[j % NBUF], out_hbm.at[idx_v.at[j]], sout[j % NBUF])
            nxt = j + NBUF
            if nxt < NCH:
                outs[j].wait()
                ins[nxt] = pltpu.async_copy(
                    src_hbm.at[pl.ds(base + nxt * GW, GW)],
                    buf.at[nxt % NBUF], sin[nxt % NBUF])
        for j in range(max(0, NCH - NBUF), NCH):
            outs[j].wait()
        tok_cp.wait()

    return k(src, slots2, slots1)


def _sc_combine(h, inv16, counts):
    scratch = [
        pltpu.VMEM((RPW, IW), I32),
        pltpu.VMEM((NCH, GW), I32),
        pltpu.VMEM((NBUF, GW, H), F32),
        pltpu.VMEM((16,), I32),
    ] + [pltpu.SemaphoreType.DMA] * (2 * NBUF)

    @functools.partial(
        pl.kernel,
        out_type=jax.ShapeDtypeStruct((DISP_ROWS, H), F32),
        mesh=_sc_mesh(),
        scratch_types=scratch,
        compiler_params=pltpu.CompilerParams(needs_layout_passes=False),
    )
    def k(h_hbm, inv_hbm, cnt_hbm, out_hbm, inv_v, idx_v, buf, cnt_v, *sems):
        sin, sout = sems[:NBUF], sems[NBUF:]
        wid = lax.axis_index("s") * 2 + lax.axis_index("c")
        base = wid * RPW
        pltpu.sync_copy(inv_hbm.at[pl.ds(base, RPW)], inv_v)
        pltpu.sync_copy(cnt_hbm, cnt_v)
        e = base // CAP
        ce = plsc.load_gather(cnt_v, [jnp.full((16,), e, I32)])
        bound = e * CAP + jnp.minimum(ce, CAP)
        zeros = jnp.zeros((16,), I32)
        lane = lax.iota(I32, 16)
        for kk in range(RPW // 16):
            s = base + kk * 16 + lane
            ids = plsc.load_gather(inv_v, [kk * 16 + lane, zeros])
            valid = s < bound
            vals = jnp.where(valid, ids, TRASH + (s & (DISP_ROWS - TRASH - 1)))
            idx_v[(kk * 16) // GW, pl.ds((kk * 16) % GW, 16)] = vals

        ins = [None] * NCH
        outs = [None] * NCH
        for j in range(min(NBUF, NCH)):
            ins[j] = pltpu.async_copy(
                h_hbm.at[pl.ds(base + j * GW, GW)], buf.at[j % NBUF],
                sin[j % NBUF])
        for j in range(NCH):
            ins[j].wait()
            outs[j] = pltpu.async_copy(
                buf.at[j % NBUF], out_hbm.at[idx_v.at[j]], sout[j % NBUF])
            nxt = j + NBUF
            if nxt < NCH:
                outs[j].wait()
                ins[nxt] = pltpu.async_copy(
                    h_hbm.at[pl.ds(base + nxt * GW, GW)],
                    buf.at[nxt % NBUF], sin[nxt % NBUF])
        for j in range(max(0, NCH - NBUF), NCH):
            outs[j].wait()

    return k(h, inv16, counts)


def kernel(x, y, W1, b1, Wg1, We1a, be1a, We1b, be1b, W2, b2, Wg2,
           We2a, be2a, We2b, be2b, W3, b3):
    xf = x.reshape(T, H)
    hidden16, logits1 = _stage1(xf, W1, b1.reshape(1, H), Wg1)

    ss1, gate1, cnt1 = _routing(logits1)
    disp1, inv1 = _sc_dispatch(hidden16, ss1.reshape(-1, GW), ss1.reshape(T))
    h1 = _ffn(disp1, We1a, be1a.reshape(E, 1, H),
              We1b, be1b.reshape(E, 1, H))
    rows1 = _sc_combine(h1, inv1, cnt1.reshape(16))

    out16, logits2 = _w2(rows1, gate1, W2, b2.reshape(1, H), Wg2)

    ss2, gate2, cnt2 = _routing(logits2)
    disp2, inv2 = _sc_dispatch(out16, ss2.reshape(-1, GW), ss2.reshape(T))
    h2 = _ffn(disp2, We2a, be2a.reshape(E, 1, H),
              We2b, be2b.reshape(E, 1, H))
    rows2 = _sc_combine(h2, inv2, cnt2.reshape(16))

    loss = _final(hidden16, rows2, gate2, W3,
                  b3.reshape(1, H), y.reshape(B, 1).astype(I32))
    return loss.reshape(())


# speedup vs baseline: 2.0238x; 1.0114x over previous
import functools

import jax
import jax.numpy as jnp
from jax import lax
from jax.experimental import pallas as pl
from jax.experimental.pallas import tpu as pltpu
from jax.experimental.pallas import tpu_sc as plsc

F32 = jnp.float32
BF16 = jnp.bfloat16
I32 = jnp.int32

T = 4096
H = 1024
E = 8
CAP = 512
B = 2
S = 2048
DISP_ROWS = 4608
TRASH = 4096
NW = 32
RPW = T // NW
GW = 32
NBUF = 3
NCH = RPW // GW
INV_ROWS = 4608
IW = 128

_BLK1 = 512
_BLKR = 256
_BLKF = 256


def _stage1_body(x_ref, w_ref, b_ref, wg_ref, hid_ref, log_ref):
    xb = x_ref[...].astype(BF16)
    h = jnp.dot(xb, w_ref[...].astype(BF16), preferred_element_type=F32)
    h = h + b_ref[...]
    hid_ref[...] = h
    log_ref[...] = jnp.dot(h.astype(BF16), wg_ref[...].astype(BF16),
                           preferred_element_type=F32)


def _routing_body(l_ref, ss_ref, g_ref, cnt_ref, carry_ref):
    pid = pl.program_id(0)

    @pl.when(pid == 0)
    def _():
        carry_ref[...] = jnp.zeros_like(carry_ref)

    l = l_ref[...]
    n = l.shape[0]
    m = jnp.max(l, axis=1, keepdims=True)
    s = jnp.sum(jnp.exp(l - m), axis=1, keepdims=True)
    gv = 1.0 / s
    ei = lax.broadcasted_iota(I32, (n, E), 1)
    idx = jnp.min(jnp.where(l == m, ei, E), axis=1, keepdims=True)
    mask = (ei == idx).astype(F32)
    ri = lax.broadcasted_iota(I32, (n, n), 0)
    ci = lax.broadcasted_iota(I32, (n, n), 1)
    tril = (ci <= ri).astype(BF16)
    incl = jnp.dot(tril, mask.astype(BF16), preferred_element_type=F32)
    incl = jnp.sum(incl * mask, axis=1, keepdims=True)
    carry = carry_ref[...]
    base = jnp.sum(carry * mask, axis=1, keepdims=True)
    carry_ref[...] = carry + jnp.sum(mask, axis=0, keepdims=True)
    loc = base + incl - 1.0
    keep = loc < CAP
    locc = jnp.minimum(loc, CAP - 1.0).astype(I32)
    slot = idx * CAP + locc
    tok = (lax.broadcasted_iota(I32, (n, 1), 0)
           + pid * n)
    trash = TRASH + (tok & (DISP_ROWS - TRASH - 1))
    ss_ref[...] = jnp.where(keep, slot, trash)
    g_ref[...] = jnp.where(keep, gv, 0.0)
    cnt = carry_ref[...].astype(I32)
    cnt_ref[...] = jnp.concatenate(
        [cnt, jnp.zeros((1, 16 - E), I32)], axis=1)


def _ffn_body(d_ref, wa_ref, ba_ref, wb_ref, bb_ref, h_ref):
    lhs = d_ref[...].astype(BF16)
    t = jnp.dot(lhs, wa_ref[0].astype(BF16), preferred_element_type=F32)
    t = t + ba_ref[0]
    h = jnp.dot(t.astype(BF16), wb_ref[0].astype(BF16),
                preferred_element_type=F32)
    h = h + bb_ref[0]
    h_ref[...] = h


def _w2_body(r_ref, g_ref, w_ref, b_ref, wg_ref, o_ref, log_ref):
    g = g_ref[...]
    lhs = jnp.where(g > 0, r_ref[...] * g, 0.0).astype(BF16)
    o = jnp.dot(lhs, w_ref[...].astype(BF16), preferred_element_type=F32)
    o = o + b_ref[...]
    o_ref[...] = o
    log_ref[...] = jnp.dot(o.astype(BF16), wg_ref[...].astype(BF16),
                           preferred_element_type=F32)


def _final_body(h_ref, r_ref, g_ref, w3_ref, b3_ref, y_ref, out_ref,
                acch_ref, acco_ref):
    pid = pl.program_id(0)

    @pl.when(pid == 0)
    def _():
        acch_ref[...] = jnp.zeros_like(acch_ref)
        acco_ref[...] = jnp.zeros_like(acco_ref)
        out_ref[...] = jnp.zeros_like(out_ref)

    b = pid // (S // _BLKF)
    rowsel = (lax.broadcasted_iota(I32, (B, 1), 0) == b).astype(F32)
    hsum = jnp.sum(h_ref[...], axis=0, keepdims=True)
    g = g_ref[...]
    osum = jnp.sum(jnp.where(g > 0, r_ref[...] * g, 0.0),
                   axis=0, keepdims=True)
    acch_ref[...] += rowsel * hsum
    acco_ref[...] += rowsel * osum

    @pl.when(pid == pl.num_programs(0) - 1)
    def _():
        sent = acch_ref[...] * (1.0 / S)
        sent = sent + jnp.dot((acco_ref[...] * (1.0 / S)).astype(BF16),
                              w3_ref[...].astype(BF16),
                              preferred_element_type=F32)
        sent = sent + b3_ref[...]
        m = jnp.max(sent, axis=1, keepdims=True)
        lz = jnp.log(jnp.sum(jnp.exp(sent - m), axis=1, keepdims=True)) + m
        ci = lax.broadcasted_iota(I32, (B, H), 1)
        picked = jnp.sum(jnp.where(ci == y_ref[...], sent, 0.0),
                         axis=1, keepdims=True)
        out_ref[...] = jnp.sum(lz - picked, axis=0, keepdims=True) / B


def _stage1(xf, W1, b1, Wg1):
    grid = (T // _BLK1,)
    return pl.pallas_call(
        _stage1_body,
        grid=grid,
        in_specs=[
            pl.BlockSpec((_BLK1, H), lambda i: (i, 0)),
            pl.BlockSpec((H, H), lambda i: (0, 0)),
            pl.BlockSpec((1, H), lambda i: (0, 0)),
            pl.BlockSpec((H, E), lambda i: (0, 0)),
        ],
        out_specs=[
            pl.BlockSpec((_BLK1, H), lambda i: (i, 0)),
            pl.BlockSpec((_BLK1, E), lambda i: (i, 0)),
        ],
        out_shape=[
            jax.ShapeDtypeStruct((T, H), F32),
            jax.ShapeDtypeStruct((T, E), F32),
        ],
        compiler_params=pltpu.CompilerParams(
            dimension_semantics=("parallel",)),
    )(xf, W1, b1, Wg1)


def _routing(logits):
    grid = (T // _BLKR,)
    return pl.pallas_call(
        _routing_body,
        grid=grid,
        in_specs=[pl.BlockSpec((_BLKR, E), lambda i: (i, 0))],
        out_specs=[
            pl.BlockSpec((_BLKR, 1), lambda i: (i, 0)),
            pl.BlockSpec((_BLKR, 1), lambda i: (i, 0)),
            pl.BlockSpec((1, 16), lambda i: (0, 0)),
        ],
        out_shape=[
            jax.ShapeDtypeStruct((T, 1), I32),
            jax.ShapeDtypeStruct((T, 1), F32),
            jax.ShapeDtypeStruct((1, 16), I32),
        ],
        scratch_shapes=[pltpu.VMEM((1, E), F32)],
        compiler_params=pltpu.CompilerParams(
            dimension_semantics=("arbitrary",)),
    )(logits)


def _ffn(disp, Wa, ba, Wb, bb):
    grid = (E,)
    return pl.pallas_call(
        _ffn_body,
        grid=grid,
        in_specs=[
            pl.BlockSpec((CAP, H), lambda e: (e, 0)),
            pl.BlockSpec((1, H, H), lambda e: (e, 0, 0)),
            pl.BlockSpec((1, 1, H), lambda e: (e, 0, 0)),
            pl.BlockSpec((1, H, H), lambda e: (e, 0, 0)),
            pl.BlockSpec((1, 1, H), lambda e: (e, 0, 0)),
        ],
        out_specs=[pl.BlockSpec((CAP, H), lambda e: (e, 0))],
        out_shape=[jax.ShapeDtypeStruct((T, H), F32)],
        compiler_params=pltpu.CompilerParams(
            dimension_semantics=("parallel",)),
    )(disp, Wa, ba, Wb, bb)[0]


def _w2(rows, gate, W2, b2, Wg2):
    grid = (T // _BLK1,)
    return pl.pallas_call(
        _w2_body,
        grid=grid,
        in_specs=[
            pl.BlockSpec((_BLK1, H), lambda i: (i, 0)),
            pl.BlockSpec((_BLK1, 1), lambda i: (i, 0)),
            pl.BlockSpec((H, H), lambda i: (0, 0)),
            pl.BlockSpec((1, H), lambda i: (0, 0)),
            pl.BlockSpec((H, E), lambda i: (0, 0)),
        ],
        out_specs=[
            pl.BlockSpec((_BLK1, H), lambda i: (i, 0)),
            pl.BlockSpec((_BLK1, E), lambda i: (i, 0)),
        ],
        out_shape=[
            jax.ShapeDtypeStruct((T, H), F32),
            jax.ShapeDtypeStruct((T, E), F32),
        ],
        compiler_params=pltpu.CompilerParams(
            dimension_semantics=("parallel",)),
    )(rows, gate, W2, b2, Wg2)


def _final(hidden, rows2, gate2, W3, b3, y2):
    grid = (T // _BLKF,)
    return pl.pallas_call(
        _final_body,
        grid=grid,
        in_specs=[
            pl.BlockSpec((_BLKF, H), lambda i: (i, 0)),
            pl.BlockSpec((_BLKF, H), lambda i: (i, 0)),
            pl.BlockSpec((_BLKF, 1), lambda i: (i, 0)),
            pl.BlockSpec((H, H), lambda i: (0, 0)),
            pl.BlockSpec((1, H), lambda i: (0, 0)),
            pl.BlockSpec((B, 1), lambda i: (0, 0)),
        ],
        out_specs=[pl.BlockSpec((1, 1), lambda i: (0, 0))],
        out_shape=[jax.ShapeDtypeStruct((1, 1), F32)],
        scratch_shapes=[pltpu.VMEM((B, H), F32), pltpu.VMEM((B, H), F32)],
        compiler_params=pltpu.CompilerParams(
            dimension_semantics=("arbitrary",)),
    )(hidden, rows2, gate2, W3, b3, y2)[0]


def _sc_mesh():
    return plsc.VectorSubcoreMesh(core_axis_name="c", subcore_axis_name="s")


def _sc_dispatch(src, slots2, slots1):
    scratch = [
        pltpu.VMEM((NCH, GW), I32),
        pltpu.VMEM((RPW,), I32),
        pltpu.VMEM((NBUF, GW, H), F32),
        pltpu.VMEM((RPW, IW), I32),
    ] + [pltpu.SemaphoreType.DMA] * (2 * NBUF + 1)

    @functools.partial(
        pl.kernel,
        out_type=[jax.ShapeDtypeStruct((DISP_ROWS, H), F32),
                  jax.ShapeDtypeStruct((INV_ROWS, IW), I32)],
        mesh=_sc_mesh(),
        scratch_types=scratch,
        compiler_params=pltpu.CompilerParams(needs_layout_passes=False),
    )
    def k(src_hbm, slot2_hbm, slot1_hbm, out_hbm, inv_hbm,
          idx_v, idxf_v, buf, tok_v, *sems):
        sin, sout, stok = sems[:NBUF], sems[NBUF:2 * NBUF], sems[2 * NBUF]
        wid = lax.axis_index("s") * 2 + lax.axis_index("c")
        base = wid * RPW
        pltpu.sync_copy(slot2_hbm.at[pl.ds(wid * NCH, NCH)], idx_v)
        pltpu.sync_copy(slot1_hbm.at[pl.ds(base, RPW)], idxf_v)
        for j in range(RPW):
            tok_v[j, pl.ds(0, 16)] = jnp.full((16,), base + j, I32)
        tok_cp = pltpu.async_copy(tok_v, inv_hbm.at[idxf_v], stok)

        ins = [None] * NCH
        outs = [None] * NCH
        for j in range(min(NBUF, NCH)):
            ins[j] = pltpu.async_copy(
                src_hbm.at[pl.ds(base + j * GW, GW)], buf.at[j % NBUF],
                sin[j % NBUF])
        for j in range(NCH):
            ins[j].wait()
            outs[j] = pltpu.async_copy(
                buf.at[j % NBUF], out_hbm.at[idx_v.at[j]], sout[j % NBUF])
            nxt = j + NBUF
            if nxt < NCH:
                outs[j].wait()
                ins[nxt] = pltpu.async_copy(
                    src_hbm.at[pl.ds(base + nxt * GW, GW)],
                    buf.at[nxt % NBUF], sin[nxt % NBUF])
        for j in range(max(0, NCH - NBUF), NCH):
            outs[j].wait()
        tok_cp.wait()

    return k(src, slots2, slots1)


def _sc_combine(h, inv16, counts):
    scratch = [
        pltpu.VMEM((RPW, IW), I32),
        pltpu.VMEM((NCH, GW), I32),
        pltpu.VMEM((NBUF, GW, H), F32),
        pltpu.VMEM((16,), I32),
    ] + [pltpu.SemaphoreType.DMA] * (2 * NBUF)

    @functools.partial(
        pl.kernel,
        out_type=jax.ShapeDtypeStruct((DISP_ROWS, H), F32),
        mesh=_sc_mesh(),
        scratch_types=scratch,
        compiler_params=pltpu.CompilerParams(needs_layout_passes=False),
    )
    def k(h_hbm, inv_hbm, cnt_hbm, out_hbm, inv_v, idx_v, buf, cnt_v, *sems):
        sin, sout = sems[:NBUF], sems[NBUF:]
        wid = lax.axis_index("s") * 2 + lax.axis_index("c")
        base = wid * RPW
        pltpu.sync_copy(inv_hbm.at[pl.ds(base, RPW)], inv_v)
        pltpu.sync_copy(cnt_hbm, cnt_v)
        e = base // CAP
        ce = plsc.load_gather(cnt_v, [jnp.full((16,), e, I32)])
        bound = e * CAP + jnp.minimum(ce, CAP)
        zeros = jnp.zeros((16,), I32)
        lane = lax.iota(I32, 16)
        for kk in range(RPW // 16):
            s = base + kk * 16 + lane
            ids = plsc.load_gather(inv_v, [kk * 16 + lane, zeros])
            valid = s < bound
            vals = jnp.where(valid, ids, TRASH + (s & (DISP_ROWS - TRASH - 1)))
            idx_v[(kk * 16) // GW, pl.ds((kk * 16) % GW, 16)] = vals

        ins = [None] * NCH
        outs = [None] * NCH
        for j in range(min(NBUF, NCH)):
            ins[j] = pltpu.async_copy(
                h_hbm.at[pl.ds(base + j * GW, GW)], buf.at[j % NBUF],
                sin[j % NBUF])
        for j in range(NCH):
            ins[j].wait()
            outs[j] = pltpu.async_copy(
                buf.at[j % NBUF], out_hbm.at[idx_v.at[j]], sout[j % NBUF])
            nxt = j + NBUF
            if nxt < NCH:
                outs[j].wait()
                ins[nxt] = pltpu.async_copy(
                    h_hbm.at[pl.ds(base + nxt * GW, GW)],
                    buf.at[nxt % NBUF], sin[nxt % NBUF])
        for j in range(max(0, NCH - NBUF), NCH):
            outs[j].wait()

    return k(h, inv16, counts)


def kernel(x, y, W1, b1, Wg1, We1a, be1a, We1b, be1b, W2, b2, Wg2,
           We2a, be2a, We2b, be2b, W3, b3):
    xf = x.reshape(T, H)
    hidden16, logits1 = _stage1(xf, W1, b1.reshape(1, H), Wg1)

    ss1, gate1, cnt1 = _routing(logits1)
    disp1, inv1 = _sc_dispatch(hidden16, ss1.reshape(-1, GW), ss1.reshape(T))
    h1 = _ffn(disp1, We1a, be1a.reshape(E, 1, H),
              We1b, be1b.reshape(E, 1, H))
    rows1 = _sc_combine(h1, inv1, cnt1.reshape(16))

    out16, logits2 = _w2(rows1, gate1, W2, b2.reshape(1, H), Wg2)

    ss2, gate2, cnt2 = _routing(logits2)
    disp2, inv2 = _sc_dispatch(out16, ss2.reshape(-1, GW), ss2.reshape(T))
    h2 = _ffn(disp2, We2a, be2a.reshape(E, 1, H),
              We2b, be2b.reshape(E, 1, H))
    rows2 = _sc_combine(h2, inv2, cnt2.reshape(16))

    loss = _final(hidden16, rows2, gate2, W3,
                  b3.reshape(1, H), y.reshape(B, 1).astype(I32))
    return loss.reshape(())


# speedup vs baseline: 2.1494x; 1.0621x over previous
import functools

import jax
import jax.numpy as jnp
from jax import lax
from jax.experimental import pallas as pl
from jax.experimental.pallas import tpu as pltpu
from jax.experimental.pallas import tpu_sc as plsc

F32 = jnp.float32
BF16 = jnp.bfloat16
I32 = jnp.int32

T = 4096
H = 1024
E = 8
CAP = 512
B = 2
S = 2048
DISP_ROWS = 4608
TRASH = 4096
NW = 32
RPW = T // NW
GW = 32
NBUF = 3
NCH = RPW // GW
INV_ROWS = 4608
IW = 128

_BLK1 = 512
_BLKR = 256
_BLKF = 256


def _stage1_body(x_ref, w_ref, b_ref, wg_ref, hid_ref, ss_ref, g_ref,
                 cnt_ref, log_v):
    pid = pl.program_id(0)
    xb = x_ref[...].astype(BF16)
    h = jnp.dot(xb, w_ref[...].astype(BF16), preferred_element_type=F32)
    h = h + b_ref[...]
    hid_ref[...] = h
    log_v[pl.ds(pid * _BLK1, _BLK1), :] = jnp.dot(
        h.astype(BF16), wg_ref[...].astype(BF16), preferred_element_type=F32)

    @pl.when(pid == pl.num_programs(0) - 1)
    def _():
        ss, gate, cnt = _routing_math(log_v[...])
        ss_ref[...] = ss
        g_ref[...] = gate
        cnt_ref[...] = cnt


def _routing_math(l):
    m = jnp.max(l, axis=1, keepdims=True)
    s = jnp.sum(jnp.exp(l - m), axis=1, keepdims=True)
    gv = 1.0 / s
    ei = lax.broadcasted_iota(I32, (T, E), 1)
    idx = jnp.min(jnp.where(l == m, ei, E), axis=1, keepdims=True)
    mask = (ei == idx).astype(F32)
    nb = 512
    ri = lax.broadcasted_iota(I32, (nb, nb), 0)
    ci = lax.broadcasted_iota(I32, (nb, nb), 1)
    tril = (ci <= ri).astype(BF16)
    carry = jnp.zeros((1, E), F32)
    locs = []
    for b in range(T // nb):
        mb = mask[b * nb:(b + 1) * nb]
        incl = jnp.dot(tril, mb.astype(BF16), preferred_element_type=F32)
        incl = jnp.sum(incl * mb, axis=1, keepdims=True)
        base = jnp.sum(carry * mb, axis=1, keepdims=True)
        locs.append(base + incl - 1.0)
        carry = carry + jnp.sum(mb, axis=0, keepdims=True)
    loc = jnp.concatenate(locs, axis=0)
    keep = loc < CAP
    locc = jnp.minimum(loc, CAP - 1.0).astype(I32)
    slot = idx * CAP + locc
    tok = lax.broadcasted_iota(I32, (T, 1), 0)
    trash = TRASH + (tok & (DISP_ROWS - TRASH - 1))
    ss = jnp.where(keep, slot, trash)
    gate = jnp.where(keep, gv, 0.0)
    cnt = jnp.concatenate(
        [carry.astype(I32), jnp.zeros((1, 16 - E), I32)], axis=1)
    return ss, gate, cnt


def _routing_body(l_ref, ss_ref, g_ref, cnt_ref, carry_ref):
    pid = pl.program_id(0)

    @pl.when(pid == 0)
    def _():
        carry_ref[...] = jnp.zeros_like(carry_ref)

    l = l_ref[...]
    n = l.shape[0]
    m = jnp.max(l, axis=1, keepdims=True)
    s = jnp.sum(jnp.exp(l - m), axis=1, keepdims=True)
    gv = 1.0 / s
    ei = lax.broadcasted_iota(I32, (n, E), 1)
    idx = jnp.min(jnp.where(l == m, ei, E), axis=1, keepdims=True)
    mask = (ei == idx).astype(F32)
    ri = lax.broadcasted_iota(I32, (n, n), 0)
    ci = lax.broadcasted_iota(I32, (n, n), 1)
    tril = (ci <= ri).astype(BF16)
    incl = jnp.dot(tril, mask.astype(BF16), preferred_element_type=F32)
    incl = jnp.sum(incl * mask, axis=1, keepdims=True)
    carry = carry_ref[...]
    base = jnp.sum(carry * mask, axis=1, keepdims=True)
    carry_ref[...] = carry + jnp.sum(mask, axis=0, keepdims=True)
    loc = base + incl - 1.0
    keep = loc < CAP
    locc = jnp.minimum(loc, CAP - 1.0).astype(I32)
    slot = idx * CAP + locc
    tok = (lax.broadcasted_iota(I32, (n, 1), 0)
           + pid * n)
    trash = TRASH + (tok & (DISP_ROWS - TRASH - 1))
    ss_ref[...] = jnp.where(keep, slot, trash)
    g_ref[...] = jnp.where(keep, gv, 0.0)
    cnt = carry_ref[...].astype(I32)
    cnt_ref[...] = jnp.concatenate(
        [cnt, jnp.zeros((1, 16 - E), I32)], axis=1)


def _ffn_body(d_ref, wa_ref, ba_ref, wb_ref, bb_ref, h_ref):
    lhs = d_ref[...].astype(BF16)
    t = jnp.dot(lhs, wa_ref[0].astype(BF16), preferred_element_type=F32)
    t = t + ba_ref[0]
    h = jnp.dot(t.astype(BF16), wb_ref[0].astype(BF16),
                preferred_element_type=F32)
    h = h + bb_ref[0]
    h_ref[...] = h


def _w2_body(r_ref, g_ref, w_ref, b_ref, wg_ref, o_ref, ss_ref, g2_ref,
             cnt_ref, log_v):
    pid = pl.program_id(0)
    g = g_ref[...]
    lhs = jnp.where(g > 0, r_ref[...] * g, 0.0).astype(BF16)
    o = jnp.dot(lhs, w_ref[...].astype(BF16), preferred_element_type=F32)
    o = o + b_ref[...]
    o_ref[...] = o
    log_v[pl.ds(pid * _BLK1, _BLK1), :] = jnp.dot(
        o.astype(BF16), wg_ref[...].astype(BF16), preferred_element_type=F32)

    @pl.when(pid == pl.num_programs(0) - 1)
    def _():
        ss, gate, cnt = _routing_math(log_v[...])
        ss_ref[...] = ss
        g2_ref[...] = gate
        cnt_ref[...] = cnt


def _final_body(h_ref, r_ref, g_ref, w3_ref, b3_ref, y_ref, out_ref,
                acch_ref, acco_ref):
    pid = pl.program_id(0)

    @pl.when(pid == 0)
    def _():
        acch_ref[...] = jnp.zeros_like(acch_ref)
        acco_ref[...] = jnp.zeros_like(acco_ref)
        out_ref[...] = jnp.zeros_like(out_ref)

    b = pid // (S // _BLKF)
    rowsel = (lax.broadcasted_iota(I32, (B, 1), 0) == b).astype(F32)
    hsum = jnp.sum(h_ref[...], axis=0, keepdims=True)
    g = g_ref[...]
    osum = jnp.sum(jnp.where(g > 0, r_ref[...] * g, 0.0),
                   axis=0, keepdims=True)
    acch_ref[...] += rowsel * hsum
    acco_ref[...] += rowsel * osum

    @pl.when(pid == pl.num_programs(0) - 1)
    def _():
        sent = acch_ref[...] * (1.0 / S)
        sent = sent + jnp.dot((acco_ref[...] * (1.0 / S)).astype(BF16),
                              w3_ref[...].astype(BF16),
                              preferred_element_type=F32)
        sent = sent + b3_ref[...]
        m = jnp.max(sent, axis=1, keepdims=True)
        lz = jnp.log(jnp.sum(jnp.exp(sent - m), axis=1, keepdims=True)) + m
        ci = lax.broadcasted_iota(I32, (B, H), 1)
        picked = jnp.sum(jnp.where(ci == y_ref[...], sent, 0.0),
                         axis=1, keepdims=True)
        out_ref[...] = jnp.sum(lz - picked, axis=0, keepdims=True) / B


def _stage1(xf, W1, b1, Wg1):
    grid = (T // _BLK1,)
    return pl.pallas_call(
        _stage1_body,
        grid=grid,
        in_specs=[
            pl.BlockSpec((_BLK1, H), lambda i: (i, 0)),
            pl.BlockSpec((H, H), lambda i: (0, 0)),
            pl.BlockSpec((1, H), lambda i: (0, 0)),
            pl.BlockSpec((H, E), lambda i: (0, 0)),
        ],
        out_specs=[
            pl.BlockSpec((_BLK1, H), lambda i: (i, 0)),
            pl.BlockSpec((T, 1), lambda i: (0, 0)),
            pl.BlockSpec((T, 1), lambda i: (0, 0)),
            pl.BlockSpec((1, 16), lambda i: (0, 0)),
        ],
        out_shape=[
            jax.ShapeDtypeStruct((T, H), F32),
            jax.ShapeDtypeStruct((T, 1), I32),
            jax.ShapeDtypeStruct((T, 1), F32),
            jax.ShapeDtypeStruct((1, 16), I32),
        ],
        scratch_shapes=[pltpu.VMEM((T, E), F32)],
        compiler_params=pltpu.CompilerParams(
            dimension_semantics=("arbitrary",)),
    )(xf, W1, b1, Wg1)


def _routing(logits):
    grid = (T // _BLKR,)
    return pl.pallas_call(
        _routing_body,
        grid=grid,
        in_specs=[pl.BlockSpec((_BLKR, E), lambda i: (i, 0))],
        out_specs=[
            pl.BlockSpec((_BLKR, 1), lambda i: (i, 0)),
            pl.BlockSpec((_BLKR, 1), lambda i: (i, 0)),
            pl.BlockSpec((1, 16), lambda i: (0, 0)),
        ],
        out_shape=[
            jax.ShapeDtypeStruct((T, 1), I32),
            jax.ShapeDtypeStruct((T, 1), F32),
            jax.ShapeDtypeStruct((1, 16), I32),
        ],
        scratch_shapes=[pltpu.VMEM((1, E), F32)],
        compiler_params=pltpu.CompilerParams(
            dimension_semantics=("arbitrary",)),
    )(logits)


def _ffn(disp, Wa, ba, Wb, bb):
    grid = (E,)
    return pl.pallas_call(
        _ffn_body,
        grid=grid,
        in_specs=[
            pl.BlockSpec((CAP, H), lambda e: (e, 0)),
            pl.BlockSpec((1, H, H), lambda e: (e, 0, 0)),
            pl.BlockSpec((1, 1, H), lambda e: (e, 0, 0)),
            pl.BlockSpec((1, H, H), lambda e: (e, 0, 0)),
            pl.BlockSpec((1, 1, H), lambda e: (e, 0, 0)),
        ],
        out_specs=[pl.BlockSpec((CAP, H), lambda e: (e, 0))],
        out_shape=[jax.ShapeDtypeStruct((T, H), F32)],
        compiler_params=pltpu.CompilerParams(
            dimension_semantics=("parallel",)),
    )(disp, Wa, ba, Wb, bb)[0]


def _w2(rows, gate, W2, b2, Wg2):
    grid = (T // _BLK1,)
    return pl.pallas_call(
        _w2_body,
        grid=grid,
        in_specs=[
            pl.BlockSpec((_BLK1, H), lambda i: (i, 0)),
            pl.BlockSpec((_BLK1, 1), lambda i: (i, 0)),
            pl.BlockSpec((H, H), lambda i: (0, 0)),
            pl.BlockSpec((1, H), lambda i: (0, 0)),
            pl.BlockSpec((H, E), lambda i: (0, 0)),
        ],
        out_specs=[
            pl.BlockSpec((_BLK1, H), lambda i: (i, 0)),
            pl.BlockSpec((T, 1), lambda i: (0, 0)),
            pl.BlockSpec((T, 1), lambda i: (0, 0)),
            pl.BlockSpec((1, 16), lambda i: (0, 0)),
        ],
        out_shape=[
            jax.ShapeDtypeStruct((T, H), F32),
            jax.ShapeDtypeStruct((T, 1), I32),
            jax.ShapeDtypeStruct((T, 1), F32),
            jax.ShapeDtypeStruct((1, 16), I32),
        ],
        scratch_shapes=[pltpu.VMEM((T, E), F32)],
        compiler_params=pltpu.CompilerParams(
            dimension_semantics=("arbitrary",)),
    )(rows, gate, W2, b2, Wg2)


def _final(hidden, rows2, gate2, W3, b3, y2):
    grid = (T // _BLKF,)
    return pl.pallas_call(
        _final_body,
        grid=grid,
        in_specs=[
            pl.BlockSpec((_BLKF, H), lambda i: (i, 0)),
            pl.BlockSpec((_BLKF, H), lambda i: (i, 0)),
            pl.BlockSpec((_BLKF, 1), lambda i: (i, 0)),
            pl.BlockSpec((H, H), lambda i: (0, 0)),
            pl.BlockSpec((1, H), lambda i: (0, 0)),
            pl.BlockSpec((B, 1), lambda i: (0, 0)),
        ],
        out_specs=[pl.BlockSpec((1, 1), lambda i: (0, 0))],
        out_shape=[jax.ShapeDtypeStruct((1, 1), F32)],
        scratch_shapes=[pltpu.VMEM((B, H), F32), pltpu.VMEM((B, H), F32)],
        compiler_params=pltpu.CompilerParams(
            dimension_semantics=("arbitrary",)),
    )(hidden, rows2, gate2, W3, b3, y2)[0]


def _sc_mesh():
    return plsc.VectorSubcoreMesh(core_axis_name="c", subcore_axis_name="s")


def _sc_dispatch(src, slots2, slots1):
    scratch = [
        pltpu.VMEM((NCH, GW), I32),
        pltpu.VMEM((RPW,), I32),
        pltpu.VMEM((NBUF, GW, H), F32),
        pltpu.VMEM((RPW, IW), I32),
    ] + [pltpu.SemaphoreType.DMA] * (2 * NBUF + 1)

    @functools.partial(
        pl.kernel,
        out_type=[jax.ShapeDtypeStruct((DISP_ROWS, H), F32),
                  jax.ShapeDtypeStruct((INV_ROWS, IW), I32)],
        mesh=_sc_mesh(),
        scratch_types=scratch,
        compiler_params=pltpu.CompilerParams(needs_layout_passes=False),
    )
    def k(src_hbm, slot2_hbm, slot1_hbm, out_hbm, inv_hbm,
          idx_v, idxf_v, buf, tok_v, *sems):
        sin, sout, stok = sems[:NBUF], sems[NBUF:2 * NBUF], sems[2 * NBUF]
        wid = lax.axis_index("s") * 2 + lax.axis_index("c")
        base = wid * RPW
        pltpu.sync_copy(slot2_hbm.at[pl.ds(wid * NCH, NCH)], idx_v)
        pltpu.sync_copy(slot1_hbm.at[pl.ds(base, RPW)], idxf_v)
        for j in range(RPW):
            tok_v[j, pl.ds(0, 16)] = jnp.full((16,), base + j, I32)
        tok_cp = pltpu.async_copy(tok_v, inv_hbm.at[idxf_v], stok)

        ins = [None] * NCH
        outs = [None] * NCH
        for j in range(min(NBUF, NCH)):
            ins[j] = pltpu.async_copy(
                src_hbm.at[pl.ds(base + j * GW, GW)], buf.at[j % NBUF],
                sin[j % NBUF])
        for j in range(NCH):
            ins[j].wait()
            outs[j] = pltpu.async_copy(
                buf.at[j % NBUF], out_hbm.at[idx_v.at[j]], sout[j % NBUF])
            nxt = j + NBUF
            if nxt < NCH:
                outs[j].wait()
                ins[nxt] = pltpu.async_copy(
                    src_hbm.at[pl.ds(base + nxt * GW, GW)],
                    buf.at[nxt % NBUF], sin[nxt % NBUF])
        for j in range(max(0, NCH - NBUF), NCH):
            outs[j].wait()
        tok_cp.wait()

    return k(src, slots2, slots1)


def _sc_combine(h, inv16, counts):
    scratch = [
        pltpu.VMEM((RPW, IW), I32),
        pltpu.VMEM((NCH, GW), I32),
        pltpu.VMEM((NBUF, GW, H), F32),
        pltpu.VMEM((16,), I32),
    ] + [pltpu.SemaphoreType.DMA] * (2 * NBUF)

    @functools.partial(
        pl.kernel,
        out_type=jax.ShapeDtypeStruct((DISP_ROWS, H), F32),
        mesh=_sc_mesh(),
        scratch_types=scratch,
        compiler_params=pltpu.CompilerParams(needs_layout_passes=False),
    )
    def k(h_hbm, inv_hbm, cnt_hbm, out_hbm, inv_v, idx_v, buf, cnt_v, *sems):
        sin, sout = sems[:NBUF], sems[NBUF:]
        wid = lax.axis_index("s") * 2 + lax.axis_index("c")
        base = wid * RPW
        pltpu.sync_copy(inv_hbm.at[pl.ds(base, RPW)], inv_v)
        pltpu.sync_copy(cnt_hbm, cnt_v)
        e = base // CAP
        ce = plsc.load_gather(cnt_v, [jnp.full((16,), e, I32)])
        bound = e * CAP + jnp.minimum(ce, CAP)
        zeros = jnp.zeros((16,), I32)
        lane = lax.iota(I32, 16)
        for kk in range(RPW // 16):
            s = base + kk * 16 + lane
            ids = plsc.load_gather(inv_v, [kk * 16 + lane, zeros])
            valid = s < bound
            vals = jnp.where(valid, ids, TRASH + (s & (DISP_ROWS - TRASH - 1)))
            idx_v[(kk * 16) // GW, pl.ds((kk * 16) % GW, 16)] = vals

        ins = [None] * NCH
        outs = [None] * NCH
        for j in range(min(NBUF, NCH)):
            ins[j] = pltpu.async_copy(
                h_hbm.at[pl.ds(base + j * GW, GW)], buf.at[j % NBUF],
                sin[j % NBUF])
        for j in range(NCH):
            ins[j].wait()
            outs[j] = pltpu.async_copy(
                buf.at[j % NBUF], out_hbm.at[idx_v.at[j]], sout[j % NBUF])
            nxt = j + NBUF
            if nxt < NCH:
                outs[j].wait()
                ins[nxt] = pltpu.async_copy(
                    h_hbm.at[pl.ds(base + nxt * GW, GW)],
                    buf.at[nxt % NBUF], sin[nxt % NBUF])
        for j in range(max(0, NCH - NBUF), NCH):
            outs[j].wait()

    return k(h, inv16, counts)


def kernel(x, y, W1, b1, Wg1, We1a, be1a, We1b, be1b, W2, b2, Wg2,
           We2a, be2a, We2b, be2b, W3, b3):
    xf = x.reshape(T, H)
    hidden16, ss1, gate1, cnt1 = _stage1(xf, W1, b1.reshape(1, H), Wg1)
    disp1, inv1 = _sc_dispatch(hidden16, ss1.reshape(-1, GW), ss1.reshape(T))
    h1 = _ffn(disp1, We1a, be1a.reshape(E, 1, H),
              We1b, be1b.reshape(E, 1, H))
    rows1 = _sc_combine(h1, inv1, cnt1.reshape(16))

    out16, ss2, gate2, cnt2 = _w2(rows1, gate1, W2, b2.reshape(1, H), Wg2)
    disp2, inv2 = _sc_dispatch(out16, ss2.reshape(-1, GW), ss2.reshape(T))
    h2 = _ffn(disp2, We2a, be2a.reshape(E, 1, H),
              We2b, be2b.reshape(E, 1, H))
    rows2 = _sc_combine(h2, inv2, cnt2.reshape(16))

    loss = _final(hidden16, rows2, gate2, W3,
                  b3.reshape(1, H), y.reshape(B, 1).astype(I32))
    return loss.reshape(())


# speedup vs baseline: 2.3535x; 1.0950x over previous
import functools

import jax
import jax.numpy as jnp
from jax import lax
from jax.experimental import pallas as pl
from jax.experimental.pallas import tpu as pltpu
from jax.experimental.pallas import tpu_sc as plsc

F32 = jnp.float32
BF16 = jnp.bfloat16
I32 = jnp.int32

T = 4096
H = 1024
E = 8
CAP = 512
B = 2
S = 2048
DISP_ROWS = 4608
TRASH = 4096
NW = 32
RPW = T // NW
GW = 32
NBUF = 3
NCH = RPW // GW
INV_ROWS = 4608
IW = 128
PW = 128

_BLK1 = 512
_BLKR = 256
_BLKF = 256


def _stage1_body(x_ref, w_ref, b_ref, wg_ref, hid_ref, ss_ref, g_ref,
                 cnt_ref, hs_ref, log_v, acc_v):
    pid = pl.program_id(0)

    @pl.when(pid == 0)
    def _():
        acc_v[...] = jnp.zeros_like(acc_v)

    xb = x_ref[...].astype(BF16)
    h = jnp.dot(xb, w_ref[...].astype(BF16), preferred_element_type=F32)
    h = h + b_ref[...]
    hid_ref[...] = h
    b = pid // ((T // B) // _BLK1)
    rowsel = (lax.broadcasted_iota(I32, (B, 1), 0) == b).astype(F32)
    acc_v[...] += rowsel * jnp.sum(h, axis=0, keepdims=True)
    log_v[pl.ds(pid * _BLK1, _BLK1), :] = jnp.dot(
        h.astype(BF16), wg_ref[...].astype(BF16), preferred_element_type=F32)

    @pl.when(pid == pl.num_programs(0) - 1)
    def _():
        ss, gate, cnt = _routing_math(log_v[...])
        ss_ref[...] = ss
        g_ref[...] = gate
        cnt_ref[...] = cnt
        hs_ref[...] = acc_v[...]


def _routing_math(l):
    m = jnp.max(l, axis=1, keepdims=True)
    s = jnp.sum(jnp.exp(l - m), axis=1, keepdims=True)
    gv = 1.0 / s
    ei = lax.broadcasted_iota(I32, (T, E), 1)
    idx = jnp.min(jnp.where(l == m, ei, E), axis=1, keepdims=True)
    mask = (ei == idx).astype(F32)
    nb = 512
    ri = lax.broadcasted_iota(I32, (nb, nb), 0)
    ci = lax.broadcasted_iota(I32, (nb, nb), 1)
    tril = (ci <= ri).astype(BF16)
    carry = jnp.zeros((1, E), F32)
    locs = []
    for b in range(T // nb):
        mb = mask[b * nb:(b + 1) * nb]
        incl = jnp.dot(tril, mb.astype(BF16), preferred_element_type=F32)
        incl = jnp.sum(incl * mb, axis=1, keepdims=True)
        base = jnp.sum(carry * mb, axis=1, keepdims=True)
        locs.append(base + incl - 1.0)
        carry = carry + jnp.sum(mb, axis=0, keepdims=True)
    loc = jnp.concatenate(locs, axis=0)
    keep = loc < CAP
    locc = jnp.minimum(loc, CAP - 1.0).astype(I32)
    slot = idx * CAP + locc
    tok = lax.broadcasted_iota(I32, (T, 1), 0)
    trash = TRASH + (tok & (DISP_ROWS - TRASH - 1))
    ss = jnp.where(keep, slot, trash)
    gate = jnp.where(keep, gv, 0.0)
    cnt = jnp.concatenate(
        [carry.astype(I32), jnp.zeros((1, 16 - E), I32)], axis=1)
    return ss, gate, cnt


def _ffn_body(d_ref, wa_ref, ba_ref, wb_ref, bb_ref, h_ref):
    lhs = d_ref[...].astype(BF16)
    t = jnp.dot(lhs, wa_ref[0].astype(BF16), preferred_element_type=F32)
    t = t + ba_ref[0]
    h = jnp.dot(t.astype(BF16), wb_ref[0].astype(BF16),
                preferred_element_type=F32)
    h = h + bb_ref[0]
    h_ref[...] = h


def _w2_body(r_ref, g_ref, w_ref, b_ref, wg_ref, o_ref, ss_ref, pay_ref,
             cnt_ref, log_v):
    pid = pl.program_id(0)
    g = g_ref[...]
    lhs = jnp.where(g > 0, r_ref[...] * g, 0.0).astype(BF16)
    o = jnp.dot(lhs, w_ref[...].astype(BF16), preferred_element_type=F32)
    o = o + b_ref[...]
    o_ref[...] = o
    log_v[pl.ds(pid * _BLK1, _BLK1), :] = jnp.dot(
        o.astype(BF16), wg_ref[...].astype(BF16), preferred_element_type=F32)

    @pl.when(pid == pl.num_programs(0) - 1)
    def _():
        ss, gate, cnt = _routing_math(log_v[...])
        ss_ref[...] = ss
        cnt_ref[...] = cnt
        tok = lax.broadcasted_iota(I32, (T, 1), 0)
        lane = lax.broadcasted_iota(I32, (T, PW), 1)
        half0 = tok < S
        sel0 = jnp.logical_and(lane == 0, half0)
        sel1 = jnp.logical_and(lane == 1, jnp.logical_not(half0))
        pay_ref[...] = jnp.where(jnp.logical_or(sel0, sel1), gate, 0.0)


def _final_body(h_ref, pay_ref, cnt_ref, hs_ref, w3_ref, b3_ref, y_ref,
                out_ref, acco_ref):
    pid = pl.program_id(0)

    @pl.when(pid == 0)
    def _():
        acco_ref[...] = jnp.zeros_like(acco_ref)
        out_ref[...] = jnp.zeros_like(out_ref)

    e = pid // (CAP // _BLKF)
    off = (pid % (CAP // _BLKF)) * _BLKF
    lane16 = lax.broadcasted_iota(I32, (1, 16), 1)
    ce = jnp.sum(jnp.where(lane16 == e, cnt_ref[...], 0), keepdims=True)
    ri = lax.broadcasted_iota(I32, (_BLKF, 1), 0)
    valid = (ri + off) < ce
    h = jnp.where(valid, h_ref[...], 0.0)
    pay = jnp.where(valid, pay_ref[...], 0.0)
    s0 = jnp.sum(pay[:, 0:1] * h, axis=0, keepdims=True)
    s1 = jnp.sum(pay[:, 1:2] * h, axis=0, keepdims=True)
    acco_ref[...] += jnp.concatenate([s0, s1], axis=0)

    @pl.when(pid == pl.num_programs(0) - 1)
    def _():
        sent = hs_ref[...] * (1.0 / S)
        sent = sent + jnp.dot((acco_ref[...] * (1.0 / S)).astype(BF16),
                              w3_ref[...].astype(BF16),
                              preferred_element_type=F32)
        sent = sent + b3_ref[...]
        m = jnp.max(sent, axis=1, keepdims=True)
        lz = jnp.log(jnp.sum(jnp.exp(sent - m), axis=1, keepdims=True)) + m
        ci = lax.broadcasted_iota(I32, (B, H), 1)
        picked = jnp.sum(jnp.where(ci == y_ref[...], sent, 0.0),
                         axis=1, keepdims=True)
        out_ref[...] = jnp.sum(lz - picked, axis=0, keepdims=True) / B


def _stage1(xf, W1, b1, Wg1):
    grid = (T // _BLK1,)
    return pl.pallas_call(
        _stage1_body,
        grid=grid,
        in_specs=[
            pl.BlockSpec((_BLK1, H), lambda i: (i, 0)),
            pl.BlockSpec((H, H), lambda i: (0, 0)),
            pl.BlockSpec((1, H), lambda i: (0, 0)),
            pl.BlockSpec((H, E), lambda i: (0, 0)),
        ],
        out_specs=[
            pl.BlockSpec((_BLK1, H), lambda i: (i, 0)),
            pl.BlockSpec((T, 1), lambda i: (0, 0)),
            pl.BlockSpec((T, 1), lambda i: (0, 0)),
            pl.BlockSpec((1, 16), lambda i: (0, 0)),
            pl.BlockSpec((B, H), lambda i: (0, 0)),
        ],
        out_shape=[
            jax.ShapeDtypeStruct((T, H), F32),
            jax.ShapeDtypeStruct((T, 1), I32),
            jax.ShapeDtypeStruct((T, 1), F32),
            jax.ShapeDtypeStruct((1, 16), I32),
            jax.ShapeDtypeStruct((B, H), F32),
        ],
        scratch_shapes=[pltpu.VMEM((T, E), F32), pltpu.VMEM((B, H), F32)],
        compiler_params=pltpu.CompilerParams(
            dimension_semantics=("arbitrary",)),
    )(xf, W1, b1, Wg1)


def _ffn(disp, Wa, ba, Wb, bb):
    grid = (E,)
    return pl.pallas_call(
        _ffn_body,
        grid=grid,
        in_specs=[
            pl.BlockSpec((CAP, H), lambda e: (e, 0)),
            pl.BlockSpec((1, H, H), lambda e: (e, 0, 0)),
            pl.BlockSpec((1, 1, H), lambda e: (e, 0, 0)),
            pl.BlockSpec((1, H, H), lambda e: (e, 0, 0)),
            pl.BlockSpec((1, 1, H), lambda e: (e, 0, 0)),
        ],
        out_specs=[pl.BlockSpec((CAP, H), lambda e: (e, 0))],
        out_shape=[jax.ShapeDtypeStruct((T, H), F32)],
        compiler_params=pltpu.CompilerParams(
            dimension_semantics=("parallel",)),
    )(disp, Wa, ba, Wb, bb)[0]


def _w2(rows, gate, W2, b2, Wg2):
    grid = (T // _BLK1,)
    return pl.pallas_call(
        _w2_body,
        grid=grid,
        in_specs=[
            pl.BlockSpec((_BLK1, H), lambda i: (i, 0)),
            pl.BlockSpec((_BLK1, 1), lambda i: (i, 0)),
            pl.BlockSpec((H, H), lambda i: (0, 0)),
            pl.BlockSpec((1, H), lambda i: (0, 0)),
            pl.BlockSpec((H, E), lambda i: (0, 0)),
        ],
        out_specs=[
            pl.BlockSpec((_BLK1, H), lambda i: (i, 0)),
            pl.BlockSpec((T, 1), lambda i: (0, 0)),
            pl.BlockSpec((T, PW), lambda i: (0, 0)),
            pl.BlockSpec((1, 16), lambda i: (0, 0)),
        ],
        out_shape=[
            jax.ShapeDtypeStruct((T, H), F32),
            jax.ShapeDtypeStruct((T, 1), I32),
            jax.ShapeDtypeStruct((T, PW), F32),
            jax.ShapeDtypeStruct((1, 16), I32),
        ],
        scratch_shapes=[pltpu.VMEM((T, E), F32)],
        compiler_params=pltpu.CompilerParams(
            dimension_semantics=("arbitrary",)),
    )(rows, gate, W2, b2, Wg2)


def _final(h2, pay_slot, cnt2, hs, W3, b3, y2):
    grid = (T // _BLKF,)
    return pl.pallas_call(
        _final_body,
        grid=grid,
        in_specs=[
            pl.BlockSpec((_BLKF, H), lambda i: (i, 0)),
            pl.BlockSpec((_BLKF, PW), lambda i: (i, 0)),
            pl.BlockSpec((1, 16), lambda i: (0, 0)),
            pl.BlockSpec((B, H), lambda i: (0, 0)),
            pl.BlockSpec((H, H), lambda i: (0, 0)),
            pl.BlockSpec((1, H), lambda i: (0, 0)),
            pl.BlockSpec((B, 1), lambda i: (0, 0)),
        ],
        out_specs=[pl.BlockSpec((1, 1), lambda i: (0, 0))],
        out_shape=[jax.ShapeDtypeStruct((1, 1), F32)],
        scratch_shapes=[pltpu.VMEM((B, H), F32)],
        compiler_params=pltpu.CompilerParams(
            dimension_semantics=("arbitrary",)),
    )(h2, pay_slot, cnt2, hs, W3, b3, y2)[0]


def _sc_mesh():
    return plsc.VectorSubcoreMesh(core_axis_name="c", subcore_axis_name="s")


def _sc_dispatch(src, slots2, slots1):
    scratch = [
        pltpu.VMEM((NCH, GW), I32),
        pltpu.VMEM((RPW,), I32),
        pltpu.VMEM((NBUF, GW, H), F32),
        pltpu.VMEM((RPW, IW), I32),
    ] + [pltpu.SemaphoreType.DMA] * (2 * NBUF + 1)

    @functools.partial(
        pl.kernel,
        out_type=[jax.ShapeDtypeStruct((DISP_ROWS, H), F32),
                  jax.ShapeDtypeStruct((INV_ROWS, IW), I32)],
        mesh=_sc_mesh(),
        scratch_types=scratch,
        compiler_params=pltpu.CompilerParams(needs_layout_passes=False),
    )
    def k(src_hbm, slot2_hbm, slot1_hbm, out_hbm, inv_hbm,
          idx_v, idxf_v, buf, tok_v, *sems):
        sin, sout, stok = sems[:NBUF], sems[NBUF:2 * NBUF], sems[2 * NBUF]
        wid = lax.axis_index("s") * 2 + lax.axis_index("c")
        base = wid * RPW
        pltpu.sync_copy(slot2_hbm.at[pl.ds(wid * NCH, NCH)], idx_v)
        pltpu.sync_copy(slot1_hbm.at[pl.ds(base, RPW)], idxf_v)
        for j in range(RPW):
            tok_v[j, pl.ds(0, 16)] = jnp.full((16,), base + j, I32)
        tok_cp = pltpu.async_copy(tok_v, inv_hbm.at[idxf_v], stok)

        ins = [None] * NCH
        outs = [None] * NCH
        for j in range(min(NBUF, NCH)):
            ins[j] = pltpu.async_copy(
                src_hbm.at[pl.ds(base + j * GW, GW)], buf.at[j % NBUF],
                sin[j % NBUF])
        for j in range(NCH):
            ins[j].wait()
            outs[j] = pltpu.async_copy(
                buf.at[j % NBUF], out_hbm.at[idx_v.at[j]], sout[j % NBUF])
            nxt = j + NBUF
            if nxt < NCH:
                outs[j].wait()
                ins[nxt] = pltpu.async_copy(
                    src_hbm.at[pl.ds(base + nxt * GW, GW)],
                    buf.at[nxt % NBUF], sin[nxt % NBUF])
        for j in range(max(0, NCH - NBUF), NCH):
            outs[j].wait()
        tok_cp.wait()

    return k(src, slots2, slots1)


def _sc_dispatch2(src, slots2, slots1, pay):
    scratch = [
        pltpu.VMEM((NCH, GW), I32),
        pltpu.VMEM((RPW,), I32),
        pltpu.VMEM((NBUF, GW, H), F32),
        pltpu.VMEM((RPW, PW), F32),
    ] + [pltpu.SemaphoreType.DMA] * (2 * NBUF + 1)

    @functools.partial(
        pl.kernel,
        out_type=[jax.ShapeDtypeStruct((DISP_ROWS, H), F32),
                  jax.ShapeDtypeStruct((INV_ROWS, PW), F32)],
        mesh=_sc_mesh(),
        scratch_types=scratch,
        compiler_params=pltpu.CompilerParams(needs_layout_passes=False),
    )
    def k(src_hbm, slot2_hbm, slot1_hbm, pay_hbm, out_hbm, pays_hbm,
          idx_v, idxf_v, buf, pay_v, *sems):
        sin, sout, spay = sems[:NBUF], sems[NBUF:2 * NBUF], sems[2 * NBUF]
        wid = lax.axis_index("s") * 2 + lax.axis_index("c")
        base = wid * RPW
        pltpu.sync_copy(slot2_hbm.at[pl.ds(wid * NCH, NCH)], idx_v)
        pltpu.sync_copy(slot1_hbm.at[pl.ds(base, RPW)], idxf_v)
        pltpu.sync_copy(pay_hbm.at[pl.ds(base, RPW)], pay_v)
        pay_cp = pltpu.async_copy(pay_v, pays_hbm.at[idxf_v], spay)

        ins = [None] * NCH
        outs = [None] * NCH
        for j in range(min(NBUF, NCH)):
            ins[j] = pltpu.async_copy(
                src_hbm.at[pl.ds(base + j * GW, GW)], buf.at[j % NBUF],
                sin[j % NBUF])
        for j in range(NCH):
            ins[j].wait()
            outs[j] = pltpu.async_copy(
                buf.at[j % NBUF], out_hbm.at[idx_v.at[j]], sout[j % NBUF])
            nxt = j + NBUF
            if nxt < NCH:
                outs[j].wait()
                ins[nxt] = pltpu.async_copy(
                    src_hbm.at[pl.ds(base + nxt * GW, GW)],
                    buf.at[nxt % NBUF], sin[nxt % NBUF])
        for j in range(max(0, NCH - NBUF), NCH):
            outs[j].wait()
        pay_cp.wait()

    return k(src, slots2, slots1, pay)


def _sc_combine(h, inv16, counts):
    scratch = [
        pltpu.VMEM((RPW, IW), I32),
        pltpu.VMEM((NCH, GW), I32),
        pltpu.VMEM((NBUF, GW, H), F32),
        pltpu.VMEM((16,), I32),
    ] + [pltpu.SemaphoreType.DMA] * (2 * NBUF)

    @functools.partial(
        pl.kernel,
        out_type=jax.ShapeDtypeStruct((DISP_ROWS, H), F32),
        mesh=_sc_mesh(),
        scratch_types=scratch,
        compiler_params=pltpu.CompilerParams(needs_layout_passes=False),
    )
    def k(h_hbm, inv_hbm, cnt_hbm, out_hbm, inv_v, idx_v, buf, cnt_v, *sems):
        sin, sout = sems[:NBUF], sems[NBUF:]
        wid = lax.axis_index("s") * 2 + lax.axis_index("c")
        base = wid * RPW
        pltpu.sync_copy(inv_hbm.at[pl.ds(base, RPW)], inv_v)
        pltpu.sync_copy(cnt_hbm, cnt_v)
        e = base // CAP
        ce = plsc.load_gather(cnt_v, [jnp.full((16,), e, I32)])
        bound = e * CAP + jnp.minimum(ce, CAP)
        zeros = jnp.zeros((16,), I32)
        lane = lax.iota(I32, 16)
        for kk in range(RPW // 16):
            s = base + kk * 16 + lane
            ids = plsc.load_gather(inv_v, [kk * 16 + lane, zeros])
            valid = s < bound
            vals = jnp.where(valid, ids, TRASH + (s & (DISP_ROWS - TRASH - 1)))
            idx_v[(kk * 16) // GW, pl.ds((kk * 16) % GW, 16)] = vals

        ins = [None] * NCH
        outs = [None] * NCH
        for j in range(min(NBUF, NCH)):
            ins[j] = pltpu.async_copy(
                h_hbm.at[pl.ds(base + j * GW, GW)], buf.at[j % NBUF],
                sin[j % NBUF])
        for j in range(NCH):
            ins[j].wait()
            outs[j] = pltpu.async_copy(
                buf.at[j % NBUF], out_hbm.at[idx_v.at[j]], sout[j % NBUF])
            nxt = j + NBUF
            if nxt < NCH:
                outs[j].wait()
                ins[nxt] = pltpu.async_copy(
                    h_hbm.at[pl.ds(base + nxt * GW, GW)],
                    buf.at[nxt % NBUF], sin[nxt % NBUF])
        for j in range(max(0, NCH - NBUF), NCH):
            outs[j].wait()

    return k(h, inv16, counts)


def kernel(x, y, W1, b1, Wg1, We1a, be1a, We1b, be1b, W2, b2, Wg2,
           We2a, be2a, We2b, be2b, W3, b3):
    xf = x.reshape(T, H)
    hidden16, ss1, gate1, cnt1, hs = _stage1(xf, W1, b1.reshape(1, H), Wg1)
    disp1, inv1 = _sc_dispatch(hidden16, ss1.reshape(-1, GW), ss1.reshape(T))
    h1 = _ffn(disp1, We1a, be1a.reshape(E, 1, H),
              We1b, be1b.reshape(E, 1, H))
    rows1 = _sc_combine(h1, inv1, cnt1.reshape(16))

    out16, ss2, pay, cnt2 = _w2(rows1, gate1, W2, b2.reshape(1, H), Wg2)
    disp2, pay_slot = _sc_dispatch2(out16, ss2.reshape(-1, GW),
                                    ss2.reshape(T), pay)
    h2 = _ffn(disp2, We2a, be2a.reshape(E, 1, H),
              We2b, be2b.reshape(E, 1, H))

    loss = _final(h2, pay_slot, cnt2, hs, W3,
                  b3.reshape(1, H), y.reshape(B, 1).astype(I32))
    return loss.reshape(())


# speedup vs baseline: 2.5345x; 1.0769x over previous
import functools

import jax
import jax.numpy as jnp
from jax import lax
from jax.experimental import pallas as pl
from jax.experimental.pallas import tpu as pltpu
from jax.experimental.pallas import tpu_sc as plsc

F32 = jnp.float32
BF16 = jnp.bfloat16
I32 = jnp.int32

T = 4096
H = 1024
E = 8
CAP = 512
B = 2
S = 2048
DISP_ROWS = 4608
TRASH = 4096
NW = 32
RPW = T // NW
GW = 32
NBUF = 3
NCH = RPW // GW
INV_ROWS = 4608
IW = 128
PW = 128

_BLK1 = 512
_BLKR = 256
_BLKF = 256


def _stage1_body(x_ref, w_ref, b_ref, wg_ref, hid_ref, ss_ref, g_ref,
                 cnt_ref, hs_ref, log_v, acc_v):
    pid = pl.program_id(0)

    @pl.when(pid == 0)
    def _():
        acc_v[...] = jnp.zeros_like(acc_v)

    xb = x_ref[...].astype(BF16)
    h = jnp.dot(xb, w_ref[...].astype(BF16), preferred_element_type=F32)
    h = h + b_ref[...]
    hid_ref[...] = h
    b = pid // ((T // B) // _BLK1)
    rowsel = (lax.broadcasted_iota(I32, (B, 1), 0) == b).astype(F32)
    acc_v[...] += rowsel * jnp.sum(h, axis=0, keepdims=True)
    log_v[pl.ds(pid * _BLK1, _BLK1), :] = jnp.dot(
        h.astype(BF16), wg_ref[...].astype(BF16), preferred_element_type=F32)

    @pl.when(pid == pl.num_programs(0) - 1)
    def _():
        ss, gate, cnt = _routing_math(log_v[...])
        ss_ref[...] = ss
        g_ref[...] = gate
        cnt_ref[...] = cnt
        hs_ref[...] = acc_v[...]


def _routing_math(l):
    m = jnp.max(l, axis=1, keepdims=True)
    s = jnp.sum(jnp.exp(l - m), axis=1, keepdims=True)
    gv = 1.0 / s
    ei = lax.broadcasted_iota(I32, (T, E), 1)
    idx = jnp.min(jnp.where(l == m, ei, E), axis=1, keepdims=True)
    mask = (ei == idx).astype(F32)
    nb = 512
    ri = lax.broadcasted_iota(I32, (nb, nb), 0)
    ci = lax.broadcasted_iota(I32, (nb, nb), 1)
    tril = (ci <= ri).astype(BF16)
    carry = jnp.zeros((1, E), F32)
    locs = []
    for b in range(T // nb):
        mb = mask[b * nb:(b + 1) * nb]
        incl = jnp.dot(tril, mb.astype(BF16), preferred_element_type=F32)
        incl = jnp.sum(incl * mb, axis=1, keepdims=True)
        base = jnp.sum(carry * mb, axis=1, keepdims=True)
        locs.append(base + incl - 1.0)
        carry = carry + jnp.sum(mb, axis=0, keepdims=True)
    loc = jnp.concatenate(locs, axis=0)
    keep = loc < CAP
    locc = jnp.minimum(loc, CAP - 1.0).astype(I32)
    slot = idx * CAP + locc
    tok = lax.broadcasted_iota(I32, (T, 1), 0)
    trash = TRASH + (tok & (DISP_ROWS - TRASH - 1))
    ss = jnp.where(keep, slot, trash)
    gate = jnp.where(keep, gv, 0.0)
    cnt = jnp.concatenate(
        [carry.astype(I32), jnp.zeros((1, 16 - E), I32)], axis=1)
    return ss, gate, cnt


def _ffn_body(d_ref, wa_ref, ba_ref, wb_ref, bb_ref, h_ref):
    lhs = d_ref[...].astype(BF16)
    t = jnp.dot(lhs, wa_ref[0].astype(BF16), preferred_element_type=F32)
    t = t + ba_ref[0]
    h = jnp.dot(t.astype(BF16), wb_ref[0].astype(BF16),
                preferred_element_type=F32)
    h = h + bb_ref[0]
    h_ref[...] = h


def _w2_body(r_ref, g_ref, w_ref, b_ref, wg_ref, o_ref, ss_ref, pay_ref,
             cnt_ref, log_v):
    pid = pl.program_id(0)
    g = g_ref[...]
    lhs = jnp.where(g > 0, r_ref[...] * g, 0.0).astype(BF16)
    o = jnp.dot(lhs, w_ref[...].astype(BF16), preferred_element_type=F32)
    o = o + b_ref[...]
    o_ref[...] = o
    log_v[pl.ds(pid * _BLK1, _BLK1), :] = jnp.dot(
        o.astype(BF16), wg_ref[...].astype(BF16), preferred_element_type=F32)

    @pl.when(pid == pl.num_programs(0) - 1)
    def _():
        ss, gate, cnt = _routing_math(log_v[...])
        ss_ref[...] = ss
        cnt_ref[...] = cnt
        tok = lax.broadcasted_iota(I32, (T, 1), 0)
        lane = lax.broadcasted_iota(I32, (T, PW), 1)
        half0 = tok < S
        sel0 = jnp.logical_and(lane == 0, half0)
        sel1 = jnp.logical_and(lane == 1, jnp.logical_not(half0))
        pay_ref[...] = jnp.where(jnp.logical_or(sel0, sel1), gate, 0.0)


def _ffn2_final_body(d_ref, wa_ref, ba_ref, wb_ref, bb_ref, pay_ref,
                     cnt_ref, hs_ref, w3_ref, b3_ref, y_ref, out_ref,
                     acco_ref):
    e = pl.program_id(0)

    @pl.when(e == 0)
    def _():
        acco_ref[...] = jnp.zeros_like(acco_ref)
        out_ref[...] = jnp.zeros_like(out_ref)

    lhs = d_ref[...].astype(BF16)
    t = jnp.dot(lhs, wa_ref[0].astype(BF16), preferred_element_type=F32)
    t = t + ba_ref[0]
    h = jnp.dot(t.astype(BF16), wb_ref[0].astype(BF16),
                preferred_element_type=F32)
    h = h + bb_ref[0]
    lane16 = lax.broadcasted_iota(I32, (1, 16), 1)
    ce = jnp.sum(jnp.where(lane16 == e, cnt_ref[...], 0), keepdims=True)
    ri = lax.broadcasted_iota(I32, (CAP, 1), 0)
    valid = ri < ce
    h = jnp.where(valid, h, 0.0)
    pay = jnp.where(valid, pay_ref[...], 0.0)
    s0 = jnp.sum(pay[:, 0:1] * h, axis=0, keepdims=True)
    s1 = jnp.sum(pay[:, 1:2] * h, axis=0, keepdims=True)
    acco_ref[...] += jnp.concatenate([s0, s1], axis=0)

    @pl.when(e == pl.num_programs(0) - 1)
    def _():
        sent = hs_ref[...] * (1.0 / S)
        sent = sent + jnp.dot((acco_ref[...] * (1.0 / S)).astype(BF16),
                              w3_ref[...].astype(BF16),
                              preferred_element_type=F32)
        sent = sent + b3_ref[...]
        m = jnp.max(sent, axis=1, keepdims=True)
        lz = jnp.log(jnp.sum(jnp.exp(sent - m), axis=1, keepdims=True)) + m
        ci = lax.broadcasted_iota(I32, (B, H), 1)
        picked = jnp.sum(jnp.where(ci == y_ref[...], sent, 0.0),
                         axis=1, keepdims=True)
        out_ref[...] = jnp.sum(lz - picked, axis=0, keepdims=True) / B


def _stage1(xf, W1, b1, Wg1):
    grid = (T // _BLK1,)
    return pl.pallas_call(
        _stage1_body,
        grid=grid,
        in_specs=[
            pl.BlockSpec((_BLK1, H), lambda i: (i, 0)),
            pl.BlockSpec((H, H), lambda i: (0, 0)),
            pl.BlockSpec((1, H), lambda i: (0, 0)),
            pl.BlockSpec((H, E), lambda i: (0, 0)),
        ],
        out_specs=[
            pl.BlockSpec((_BLK1, H), lambda i: (i, 0)),
            pl.BlockSpec((T, 1), lambda i: (0, 0)),
            pl.BlockSpec((T, 1), lambda i: (0, 0)),
            pl.BlockSpec((1, 16), lambda i: (0, 0)),
            pl.BlockSpec((B, H), lambda i: (0, 0)),
        ],
        out_shape=[
            jax.ShapeDtypeStruct((T, H), F32),
            jax.ShapeDtypeStruct((T, 1), I32),
            jax.ShapeDtypeStruct((T, 1), F32),
            jax.ShapeDtypeStruct((1, 16), I32),
            jax.ShapeDtypeStruct((B, H), F32),
        ],
        scratch_shapes=[pltpu.VMEM((T, E), F32), pltpu.VMEM((B, H), F32)],
        compiler_params=pltpu.CompilerParams(
            dimension_semantics=("arbitrary",)),
    )(xf, W1, b1, Wg1)


def _ffn(disp, Wa, ba, Wb, bb):
    grid = (E,)
    return pl.pallas_call(
        _ffn_body,
        grid=grid,
        in_specs=[
            pl.BlockSpec((CAP, H), lambda e: (e, 0)),
            pl.BlockSpec((1, H, H), lambda e: (e, 0, 0)),
            pl.BlockSpec((1, 1, H), lambda e: (e, 0, 0)),
            pl.BlockSpec((1, H, H), lambda e: (e, 0, 0)),
            pl.BlockSpec((1, 1, H), lambda e: (e, 0, 0)),
        ],
        out_specs=[pl.BlockSpec((CAP, H), lambda e: (e, 0))],
        out_shape=[jax.ShapeDtypeStruct((T, H), F32)],
        compiler_params=pltpu.CompilerParams(
            dimension_semantics=("parallel",)),
    )(disp, Wa, ba, Wb, bb)[0]


def _w2(rows, gate, W2, b2, Wg2):
    grid = (T // _BLK1,)
    return pl.pallas_call(
        _w2_body,
        grid=grid,
        in_specs=[
            pl.BlockSpec((_BLK1, H), lambda i: (i, 0)),
            pl.BlockSpec((_BLK1, 1), lambda i: (i, 0)),
            pl.BlockSpec((H, H), lambda i: (0, 0)),
            pl.BlockSpec((1, H), lambda i: (0, 0)),
            pl.BlockSpec((H, E), lambda i: (0, 0)),
        ],
        out_specs=[
            pl.BlockSpec((_BLK1, H), lambda i: (i, 0)),
            pl.BlockSpec((T, 1), lambda i: (0, 0)),
            pl.BlockSpec((T, PW), lambda i: (0, 0)),
            pl.BlockSpec((1, 16), lambda i: (0, 0)),
        ],
        out_shape=[
            jax.ShapeDtypeStruct((T, H), F32),
            jax.ShapeDtypeStruct((T, 1), I32),
            jax.ShapeDtypeStruct((T, PW), F32),
            jax.ShapeDtypeStruct((1, 16), I32),
        ],
        scratch_shapes=[pltpu.VMEM((T, E), F32)],
        compiler_params=pltpu.CompilerParams(
            dimension_semantics=("arbitrary",)),
    )(rows, gate, W2, b2, Wg2)


def _ffn2_final(disp, Wa, ba, Wb, bb, pay_slot, cnt2, hs, W3, b3, y2):
    grid = (E,)
    return pl.pallas_call(
        _ffn2_final_body,
        grid=grid,
        in_specs=[
            pl.BlockSpec((CAP, H), lambda e: (e, 0)),
            pl.BlockSpec((1, H, H), lambda e: (e, 0, 0)),
            pl.BlockSpec((1, 1, H), lambda e: (e, 0, 0)),
            pl.BlockSpec((1, H, H), lambda e: (e, 0, 0)),
            pl.BlockSpec((1, 1, H), lambda e: (e, 0, 0)),
            pl.BlockSpec((CAP, PW), lambda e: (e, 0)),
            pl.BlockSpec((1, 16), lambda e: (0, 0)),
            pl.BlockSpec((B, H), lambda e: (0, 0)),
            pl.BlockSpec((H, H), lambda e: (0, 0)),
            pl.BlockSpec((1, H), lambda e: (0, 0)),
            pl.BlockSpec((B, 1), lambda e: (0, 0)),
        ],
        out_specs=[pl.BlockSpec((1, 1), lambda e: (0, 0))],
        out_shape=[jax.ShapeDtypeStruct((1, 1), F32)],
        scratch_shapes=[pltpu.VMEM((B, H), F32)],
        compiler_params=pltpu.CompilerParams(
            dimension_semantics=("arbitrary",)),
    )(disp, Wa, ba, Wb, bb, pay_slot, cnt2, hs, W3, b3, y2)[0]


def _sc_mesh():
    return plsc.VectorSubcoreMesh(core_axis_name="c", subcore_axis_name="s")


def _sc_dispatch(src, slots2, slots1):
    scratch = [
        pltpu.VMEM((NCH, GW), I32),
        pltpu.VMEM((RPW,), I32),
        pltpu.VMEM((NBUF, GW, H), F32),
        pltpu.VMEM((RPW, IW), I32),
    ] + [pltpu.SemaphoreType.DMA] * (2 * NBUF + 1)

    @functools.partial(
        pl.kernel,
        out_type=[jax.ShapeDtypeStruct((DISP_ROWS, H), F32),
                  jax.ShapeDtypeStruct((INV_ROWS, IW), I32)],
        mesh=_sc_mesh(),
        scratch_types=scratch,
        compiler_params=pltpu.CompilerParams(needs_layout_passes=False),
    )
    def k(src_hbm, slot2_hbm, slot1_hbm, out_hbm, inv_hbm,
          idx_v, idxf_v, buf, tok_v, *sems):
        sin, sout, stok = sems[:NBUF], sems[NBUF:2 * NBUF], sems[2 * NBUF]
        wid = lax.axis_index("s") * 2 + lax.axis_index("c")
        base = wid * RPW
        pltpu.sync_copy(slot2_hbm.at[pl.ds(wid * NCH, NCH)], idx_v)
        pltpu.sync_copy(slot1_hbm.at[pl.ds(base, RPW)], idxf_v)
        for j in range(RPW):
            tok_v[j, pl.ds(0, 16)] = jnp.full((16,), base + j, I32)
        tok_cp = pltpu.async_copy(tok_v, inv_hbm.at[idxf_v], stok)

        ins = [None] * NCH
        outs = [None] * NCH
        for j in range(min(NBUF, NCH)):
            ins[j] = pltpu.async_copy(
                src_hbm.at[pl.ds(base + j * GW, GW)], buf.at[j % NBUF],
                sin[j % NBUF])
        for j in range(NCH):
            ins[j].wait()
            outs[j] = pltpu.async_copy(
                buf.at[j % NBUF], out_hbm.at[idx_v.at[j]], sout[j % NBUF])
            nxt = j + NBUF
            if nxt < NCH:
                outs[j].wait()
                ins[nxt] = pltpu.async_copy(
                    src_hbm.at[pl.ds(base + nxt * GW, GW)],
                    buf.at[nxt % NBUF], sin[nxt % NBUF])
        for j in range(max(0, NCH - NBUF), NCH):
            outs[j].wait()
        tok_cp.wait()

    return k(src, slots2, slots1)


def _sc_dispatch2(src, slots2, slots1, pay):
    scratch = [
        pltpu.VMEM((NCH, GW), I32),
        pltpu.VMEM((RPW,), I32),
        pltpu.VMEM((NBUF, GW, H), F32),
        pltpu.VMEM((RPW, PW), F32),
    ] + [pltpu.SemaphoreType.DMA] * (2 * NBUF + 1)

    @functools.partial(
        pl.kernel,
        out_type=[jax.ShapeDtypeStruct((DISP_ROWS, H), F32),
                  jax.ShapeDtypeStruct((INV_ROWS, PW), F32)],
        mesh=_sc_mesh(),
        scratch_types=scratch,
        compiler_params=pltpu.CompilerParams(needs_layout_passes=False),
    )
    def k(src_hbm, slot2_hbm, slot1_hbm, pay_hbm, out_hbm, pays_hbm,
          idx_v, idxf_v, buf, pay_v, *sems):
        sin, sout, spay = sems[:NBUF], sems[NBUF:2 * NBUF], sems[2 * NBUF]
        wid = lax.axis_index("s") * 2 + lax.axis_index("c")
        base = wid * RPW
        pltpu.sync_copy(slot2_hbm.at[pl.ds(wid * NCH, NCH)], idx_v)
        pltpu.sync_copy(slot1_hbm.at[pl.ds(base, RPW)], idxf_v)
        pltpu.sync_copy(pay_hbm.at[pl.ds(base, RPW)], pay_v)
        pay_cp = pltpu.async_copy(pay_v, pays_hbm.at[idxf_v], spay)

        ins = [None] * NCH
        outs = [None] * NCH
        for j in range(min(NBUF, NCH)):
            ins[j] = pltpu.async_copy(
                src_hbm.at[pl.ds(base + j * GW, GW)], buf.at[j % NBUF],
                sin[j % NBUF])
        for j in range(NCH):
            ins[j].wait()
            outs[j] = pltpu.async_copy(
                buf.at[j % NBUF], out_hbm.at[idx_v.at[j]], sout[j % NBUF])
            nxt = j + NBUF
            if nxt < NCH:
                outs[j].wait()
                ins[nxt] = pltpu.async_copy(
                    src_hbm.at[pl.ds(base + nxt * GW, GW)],
                    buf.at[nxt % NBUF], sin[nxt % NBUF])
        for j in range(max(0, NCH - NBUF), NCH):
            outs[j].wait()
        pay_cp.wait()

    return k(src, slots2, slots1, pay)


def _sc_combine(h, inv16, counts):
    scratch = [
        pltpu.VMEM((RPW, IW), I32),
        pltpu.VMEM((NCH, GW), I32),
        pltpu.VMEM((NBUF, GW, H), F32),
        pltpu.VMEM((16,), I32),
    ] + [pltpu.SemaphoreType.DMA] * (2 * NBUF)

    @functools.partial(
        pl.kernel,
        out_type=jax.ShapeDtypeStruct((DISP_ROWS, H), F32),
        mesh=_sc_mesh(),
        scratch_types=scratch,
        compiler_params=pltpu.CompilerParams(needs_layout_passes=False),
    )
    def k(h_hbm, inv_hbm, cnt_hbm, out_hbm, inv_v, idx_v, buf, cnt_v, *sems):
        sin, sout = sems[:NBUF], sems[NBUF:]
        wid = lax.axis_index("s") * 2 + lax.axis_index("c")
        base = wid * RPW
        pltpu.sync_copy(inv_hbm.at[pl.ds(base, RPW)], inv_v)
        pltpu.sync_copy(cnt_hbm, cnt_v)
        e = base // CAP
        ce = plsc.load_gather(cnt_v, [jnp.full((16,), e, I32)])
        bound = e * CAP + jnp.minimum(ce, CAP)
        zeros = jnp.zeros((16,), I32)
        lane = lax.iota(I32, 16)
        for kk in range(RPW // 16):
            s = base + kk * 16 + lane
            ids = plsc.load_gather(inv_v, [kk * 16 + lane, zeros])
            valid = s < bound
            vals = jnp.where(valid, ids, TRASH + (s & (DISP_ROWS - TRASH - 1)))
            idx_v[(kk * 16) // GW, pl.ds((kk * 16) % GW, 16)] = vals

        ins = [None] * NCH
        outs = [None] * NCH
        for j in range(min(NBUF, NCH)):
            ins[j] = pltpu.async_copy(
                h_hbm.at[pl.ds(base + j * GW, GW)], buf.at[j % NBUF],
                sin[j % NBUF])
        for j in range(NCH):
            ins[j].wait()
            outs[j] = pltpu.async_copy(
                buf.at[j % NBUF], out_hbm.at[idx_v.at[j]], sout[j % NBUF])
            nxt = j + NBUF
            if nxt < NCH:
                outs[j].wait()
                ins[nxt] = pltpu.async_copy(
                    h_hbm.at[pl.ds(base + nxt * GW, GW)],
                    buf.at[nxt % NBUF], sin[nxt % NBUF])
        for j in range(max(0, NCH - NBUF), NCH):
            outs[j].wait()

    return k(h, inv16, counts)


def kernel(x, y, W1, b1, Wg1, We1a, be1a, We1b, be1b, W2, b2, Wg2,
           We2a, be2a, We2b, be2b, W3, b3):
    xf = x.reshape(T, H)
    hidden16, ss1, gate1, cnt1, hs = _stage1(xf, W1, b1.reshape(1, H), Wg1)
    disp1, inv1 = _sc_dispatch(hidden16, ss1.reshape(-1, GW), ss1.reshape(T))
    h1 = _ffn(disp1, We1a, be1a.reshape(E, 1, H),
              We1b, be1b.reshape(E, 1, H))
    rows1 = _sc_combine(h1, inv1, cnt1.reshape(16))

    out16, ss2, pay, cnt2 = _w2(rows1, gate1, W2, b2.reshape(1, H), Wg2)
    disp2, pay_slot = _sc_dispatch2(out16, ss2.reshape(-1, GW),
                                    ss2.reshape(T), pay)
    loss = _ffn2_final(disp2, We2a, be2a.reshape(E, 1, H),
                       We2b, be2b.reshape(E, 1, H), pay_slot, cnt2, hs,
                       W3, b3.reshape(1, H), y.reshape(B, 1).astype(I32))
    return loss.reshape(())


# speedup vs baseline: 2.5493x; 1.0058x over previous
import functools

import jax
import jax.numpy as jnp
from jax import lax
from jax.experimental import pallas as pl
from jax.experimental.pallas import tpu as pltpu
from jax.experimental.pallas import tpu_sc as plsc

F32 = jnp.float32
BF16 = jnp.bfloat16
I32 = jnp.int32

T = 4096
H = 1024
E = 8
CAP = 512
B = 2
S = 2048
DISP_ROWS = 4608
TRASH = 4096
NW = 32
RPW = T // NW
GW = 32
NBUF = 3
NCH = RPW // GW
INV_ROWS = 4608
IW = 128
PW = 128

_BLK1 = 512
_BLKR = 256
_BLKF = 256


def _stage1_body(x_ref, w_ref, b_ref, wg_ref, hid_ref, ss_ref, g_ref,
                 cnt_ref, hs_ref, log_v, acc_v):
    pid = pl.program_id(0)

    @pl.when(pid == 0)
    def _():
        acc_v[...] = jnp.zeros_like(acc_v)

    xb = x_ref[...].astype(BF16)
    h = jnp.dot(xb, w_ref[...].astype(BF16), preferred_element_type=F32)
    h = h + b_ref[...]
    hid_ref[...] = h
    b = pid // ((T // B) // _BLK1)
    rowsel = (lax.broadcasted_iota(I32, (B, 1), 0) == b).astype(F32)
    acc_v[...] += rowsel * jnp.sum(h, axis=0, keepdims=True)
    log_v[pl.ds(pid * _BLK1, _BLK1), :] = jnp.dot(
        h.astype(BF16), wg_ref[...].astype(BF16), preferred_element_type=F32)

    @pl.when(pid == pl.num_programs(0) - 1)
    def _():
        ss, gate, cnt = _routing_math(log_v[...])
        ss_ref[...] = ss
        g_ref[...] = gate
        cnt_ref[...] = cnt
        hs_ref[...] = acc_v[...]


def _routing_math(l):
    m = jnp.max(l, axis=1, keepdims=True)
    s = jnp.sum(jnp.exp(l - m), axis=1, keepdims=True)
    gv = 1.0 / s
    ei = lax.broadcasted_iota(I32, (T, E), 1)
    idx = jnp.min(jnp.where(l == m, ei, E), axis=1, keepdims=True)
    mask = (ei == idx).astype(F32)
    nb = 512
    ri = lax.broadcasted_iota(I32, (nb, nb), 0)
    ci = lax.broadcasted_iota(I32, (nb, nb), 1)
    tril = (ci <= ri).astype(BF16)
    carry = jnp.zeros((1, E), F32)
    locs = []
    for b in range(T // nb):
        mb = mask[b * nb:(b + 1) * nb]
        incl = jnp.dot(tril, mb.astype(BF16), preferred_element_type=F32)
        incl = jnp.sum(incl * mb, axis=1, keepdims=True)
        base = jnp.sum(carry * mb, axis=1, keepdims=True)
        locs.append(base + incl - 1.0)
        carry = carry + jnp.sum(mb, axis=0, keepdims=True)
    loc = jnp.concatenate(locs, axis=0)
    keep = loc < CAP
    locc = jnp.minimum(loc, CAP - 1.0).astype(I32)
    slot = idx * CAP + locc
    tok = lax.broadcasted_iota(I32, (T, 1), 0)
    trash = TRASH + (tok & (DISP_ROWS - TRASH - 1))
    ss = jnp.where(keep, slot, trash)
    gate = jnp.where(keep, gv, 0.0)
    cnt = jnp.concatenate(
        [carry.astype(I32), jnp.zeros((1, 16 - E), I32)], axis=1)
    return ss, gate, cnt


def _ffn_body(d_ref, wa_ref, ba_ref, wb_ref, bb_ref, h_ref):
    lhs = d_ref[...].astype(BF16)
    t = jnp.dot(lhs, wa_ref[0].astype(BF16), preferred_element_type=F32)
    t = t + ba_ref[0]
    h = jnp.dot(t.astype(BF16), wb_ref[0].astype(BF16),
                preferred_element_type=F32)
    h = h + bb_ref[0]
    h_ref[...] = h


def _w2_body(r_ref, g_ref, w_ref, b_ref, wg_ref, o_ref, ss_ref, pay_ref,
             cnt_ref, log_v):
    pid = pl.program_id(0)
    g = g_ref[...]
    lhs = jnp.where(g > 0, r_ref[...] * g, 0.0).astype(BF16)
    o = jnp.dot(lhs, w_ref[...].astype(BF16), preferred_element_type=F32)
    o = o + b_ref[...]
    o_ref[...] = o
    log_v[pl.ds(pid * _BLK1, _BLK1), :] = jnp.dot(
        o.astype(BF16), wg_ref[...].astype(BF16), preferred_element_type=F32)

    @pl.when(pid == pl.num_programs(0) - 1)
    def _():
        ss, gate, cnt = _routing_math(log_v[...])
        ss_ref[...] = ss
        cnt_ref[...] = cnt
        tok = lax.broadcasted_iota(I32, (T, 1), 0)
        lane = lax.broadcasted_iota(I32, (T, PW), 1)
        half0 = tok < S
        sel0 = jnp.logical_and(lane == 0, half0)
        sel1 = jnp.logical_and(lane == 1, jnp.logical_not(half0))
        pay_ref[...] = jnp.where(jnp.logical_or(sel0, sel1), gate, 0.0)


def _ffn2_final_body(d_ref, wa_ref, ba_ref, wb_ref, bb_ref, pay_ref,
                     cnt_ref, hs_ref, w3_ref, b3_ref, y_ref, out_ref,
                     acco_ref):
    e = pl.program_id(0)

    @pl.when(e == 0)
    def _():
        acco_ref[...] = jnp.zeros_like(acco_ref)
        out_ref[...] = jnp.zeros_like(out_ref)

    lhs = d_ref[...].astype(BF16)
    t = jnp.dot(lhs, wa_ref[0].astype(BF16), preferred_element_type=F32)
    t = t + ba_ref[0]
    h = jnp.dot(t.astype(BF16), wb_ref[0].astype(BF16),
                preferred_element_type=F32)
    h = h + bb_ref[0]
    lane16 = lax.broadcasted_iota(I32, (1, 16), 1)
    ce = jnp.sum(jnp.where(lane16 == e, cnt_ref[...], 0), keepdims=True)
    ri = lax.broadcasted_iota(I32, (CAP, 1), 0)
    valid = ri < ce
    h = jnp.where(valid, h, 0.0)
    pay = jnp.where(valid, pay_ref[...], 0.0)
    s0 = jnp.sum(pay[:, 0:1] * h, axis=0, keepdims=True)
    s1 = jnp.sum(pay[:, 1:2] * h, axis=0, keepdims=True)
    acco_ref[...] += jnp.concatenate([s0, s1], axis=0)

    @pl.when(e == pl.num_programs(0) - 1)
    def _():
        sent = hs_ref[...] * (1.0 / S)
        sent = sent + jnp.dot((acco_ref[...] * (1.0 / S)).astype(BF16),
                              w3_ref[...].astype(BF16),
                              preferred_element_type=F32)
        sent = sent + b3_ref[...]
        m = jnp.max(sent, axis=1, keepdims=True)
        lz = jnp.log(jnp.sum(jnp.exp(sent - m), axis=1, keepdims=True)) + m
        ci = lax.broadcasted_iota(I32, (B, H), 1)
        picked = jnp.sum(jnp.where(ci == y_ref[...], sent, 0.0),
                         axis=1, keepdims=True)
        out_ref[...] = jnp.sum(lz - picked, axis=0, keepdims=True) / B


def _stage1(xf, W1, b1, Wg1):
    grid = (T // _BLK1,)
    return pl.pallas_call(
        _stage1_body,
        grid=grid,
        in_specs=[
            pl.BlockSpec((_BLK1, H), lambda i: (i, 0)),
            pl.BlockSpec((H, H), lambda i: (0, 0)),
            pl.BlockSpec((1, H), lambda i: (0, 0)),
            pl.BlockSpec((H, E), lambda i: (0, 0)),
        ],
        out_specs=[
            pl.BlockSpec((_BLK1, H), lambda i: (i, 0)),
            pl.BlockSpec((T, 1), lambda i: (0, 0)),
            pl.BlockSpec((T, 1), lambda i: (0, 0)),
            pl.BlockSpec((1, 16), lambda i: (0, 0)),
            pl.BlockSpec((B, H), lambda i: (0, 0)),
        ],
        out_shape=[
            jax.ShapeDtypeStruct((T, H), F32),
            jax.ShapeDtypeStruct((T, 1), I32),
            jax.ShapeDtypeStruct((T, 1), F32),
            jax.ShapeDtypeStruct((1, 16), I32),
            jax.ShapeDtypeStruct((B, H), F32),
        ],
        scratch_shapes=[pltpu.VMEM((T, E), F32), pltpu.VMEM((B, H), F32)],
        compiler_params=pltpu.CompilerParams(
            dimension_semantics=("arbitrary",)),
    )(xf, W1, b1, Wg1)


def _ffn(disp, Wa, ba, Wb, bb):
    grid = (E,)
    return pl.pallas_call(
        _ffn_body,
        grid=grid,
        in_specs=[
            pl.BlockSpec((CAP, H), lambda e: (e, 0)),
            pl.BlockSpec((1, H, H), lambda e: (e, 0, 0)),
            pl.BlockSpec((1, 1, H), lambda e: (e, 0, 0)),
            pl.BlockSpec((1, H, H), lambda e: (e, 0, 0)),
            pl.BlockSpec((1, 1, H), lambda e: (e, 0, 0)),
        ],
        out_specs=[pl.BlockSpec((CAP, H), lambda e: (e, 0))],
        out_shape=[jax.ShapeDtypeStruct((T, H), F32)],
        compiler_params=pltpu.CompilerParams(
            dimension_semantics=("parallel",)),
    )(disp, Wa, ba, Wb, bb)[0]


def _w2(rows, gate, W2, b2, Wg2):
    grid = (T // _BLK1,)
    return pl.pallas_call(
        _w2_body,
        grid=grid,
        in_specs=[
            pl.BlockSpec((_BLK1, H), lambda i: (i, 0)),
            pl.BlockSpec((_BLK1, 1), lambda i: (i, 0)),
            pl.BlockSpec((H, H), lambda i: (0, 0)),
            pl.BlockSpec((1, H), lambda i: (0, 0)),
            pl.BlockSpec((H, E), lambda i: (0, 0)),
        ],
        out_specs=[
            pl.BlockSpec((_BLK1, H), lambda i: (i, 0)),
            pl.BlockSpec((T, 1), lambda i: (0, 0)),
            pl.BlockSpec((T, PW), lambda i: (0, 0)),
            pl.BlockSpec((1, 16), lambda i: (0, 0)),
        ],
        out_shape=[
            jax.ShapeDtypeStruct((T, H), F32),
            jax.ShapeDtypeStruct((T, 1), I32),
            jax.ShapeDtypeStruct((T, PW), F32),
            jax.ShapeDtypeStruct((1, 16), I32),
        ],
        scratch_shapes=[pltpu.VMEM((T, E), F32)],
        compiler_params=pltpu.CompilerParams(
            dimension_semantics=("arbitrary",)),
    )(rows, gate, W2, b2, Wg2)


def _ffn2_final(disp, Wa, ba, Wb, bb, pay_slot, cnt2, hs, W3, b3, y2):
    grid = (E,)
    return pl.pallas_call(
        _ffn2_final_body,
        grid=grid,
        in_specs=[
            pl.BlockSpec((CAP, H), lambda e: (e, 0)),
            pl.BlockSpec((1, H, H), lambda e: (e, 0, 0)),
            pl.BlockSpec((1, 1, H), lambda e: (e, 0, 0)),
            pl.BlockSpec((1, H, H), lambda e: (e, 0, 0)),
            pl.BlockSpec((1, 1, H), lambda e: (e, 0, 0)),
            pl.BlockSpec((CAP, PW), lambda e: (e, 0)),
            pl.BlockSpec((1, 16), lambda e: (0, 0)),
            pl.BlockSpec((B, H), lambda e: (0, 0)),
            pl.BlockSpec((H, H), lambda e: (0, 0)),
            pl.BlockSpec((1, H), lambda e: (0, 0)),
            pl.BlockSpec((B, 1), lambda e: (0, 0)),
        ],
        out_specs=[pl.BlockSpec((1, 1), lambda e: (0, 0))],
        out_shape=[jax.ShapeDtypeStruct((1, 1), F32)],
        scratch_shapes=[pltpu.VMEM((B, H), F32)],
        compiler_params=pltpu.CompilerParams(
            dimension_semantics=("arbitrary",)),
    )(disp, Wa, ba, Wb, bb, pay_slot, cnt2, hs, W3, b3, y2)[0]


def _sc_mesh():
    return plsc.VectorSubcoreMesh(core_axis_name="c", subcore_axis_name="s")


def _sc_dispatch(src, slots2):
    scratch = [
        pltpu.VMEM((NCH, GW), I32),
        pltpu.VMEM((NBUF, GW, H), F32),
        pltpu.VMEM((RPW, IW), I32),
    ] + [pltpu.SemaphoreType.DMA] * (2 * NBUF + 1)

    @functools.partial(
        pl.kernel,
        out_type=[jax.ShapeDtypeStruct((DISP_ROWS, H), F32),
                  jax.ShapeDtypeStruct((INV_ROWS, IW), I32)],
        mesh=_sc_mesh(),
        scratch_types=scratch,
        compiler_params=pltpu.CompilerParams(needs_layout_passes=False),
    )
    def k(src_hbm, slot2_hbm, out_hbm, inv_hbm, idx_v, buf, tok_v, *sems):
        sin, sout, stok = sems[:NBUF], sems[NBUF:2 * NBUF], sems[2 * NBUF]
        wid = lax.axis_index("s") * 2 + lax.axis_index("c")
        base = wid * RPW
        pltpu.sync_copy(slot2_hbm.at[pl.ds(wid * NCH, NCH)], idx_v)
        for j in range(RPW):
            tok_v[j, pl.ds(0, 16)] = jnp.full((16,), base + j, I32)
        tok_cps = [
            pltpu.async_copy(tok_v.at[pl.ds(j * GW, GW)],
                             inv_hbm.at[idx_v.at[j]], stok)
            for j in range(NCH)]

        ins = [None] * NCH
        outs = [None] * NCH
        for j in range(min(NBUF, NCH)):
            ins[j] = pltpu.async_copy(
                src_hbm.at[pl.ds(base + j * GW, GW)], buf.at[j % NBUF],
                sin[j % NBUF])
        for j in range(NCH):
            ins[j].wait()
            outs[j] = pltpu.async_copy(
                buf.at[j % NBUF], out_hbm.at[idx_v.at[j]], sout[j % NBUF])
            nxt = j + NBUF
            if nxt < NCH:
                outs[j].wait()
                ins[nxt] = pltpu.async_copy(
                    src_hbm.at[pl.ds(base + nxt * GW, GW)],
                    buf.at[nxt % NBUF], sin[nxt % NBUF])
        for j in range(max(0, NCH - NBUF), NCH):
            outs[j].wait()
        for cp in tok_cps:
            cp.wait()

    return k(src, slots2)


def _sc_dispatch2(src, slots2, pay):
    scratch = [
        pltpu.VMEM((NCH, GW), I32),
        pltpu.VMEM((NBUF, GW, H), F32),
        pltpu.VMEM((RPW, PW), F32),
    ] + [pltpu.SemaphoreType.DMA] * (2 * NBUF + 1)

    @functools.partial(
        pl.kernel,
        out_type=[jax.ShapeDtypeStruct((DISP_ROWS, H), F32),
                  jax.ShapeDtypeStruct((INV_ROWS, PW), F32)],
        mesh=_sc_mesh(),
        scratch_types=scratch,
        compiler_params=pltpu.CompilerParams(needs_layout_passes=False),
    )
    def k(src_hbm, slot2_hbm, pay_hbm, out_hbm, pays_hbm,
          idx_v, buf, pay_v, *sems):
        sin, sout, spay = sems[:NBUF], sems[NBUF:2 * NBUF], sems[2 * NBUF]
        wid = lax.axis_index("s") * 2 + lax.axis_index("c")
        base = wid * RPW
        pltpu.sync_copy(slot2_hbm.at[pl.ds(wid * NCH, NCH)], idx_v)
        pltpu.sync_copy(pay_hbm.at[pl.ds(base, RPW)], pay_v)
        pay_cps = [
            pltpu.async_copy(pay_v.at[pl.ds(j * GW, GW)],
                             pays_hbm.at[idx_v.at[j]], spay)
            for j in range(NCH)]

        ins = [None] * NCH
        outs = [None] * NCH
        for j in range(min(NBUF, NCH)):
            ins[j] = pltpu.async_copy(
                src_hbm.at[pl.ds(base + j * GW, GW)], buf.at[j % NBUF],
                sin[j % NBUF])
        for j in range(NCH):
            ins[j].wait()
            outs[j] = pltpu.async_copy(
                buf.at[j % NBUF], out_hbm.at[idx_v.at[j]], sout[j % NBUF])
            nxt = j + NBUF
            if nxt < NCH:
                outs[j].wait()
                ins[nxt] = pltpu.async_copy(
                    src_hbm.at[pl.ds(base + nxt * GW, GW)],
                    buf.at[nxt % NBUF], sin[nxt % NBUF])
        for j in range(max(0, NCH - NBUF), NCH):
            outs[j].wait()
        for cp in pay_cps:
            cp.wait()

    return k(src, slots2, pay)


def _sc_combine(h, inv16, counts):
    scratch = [
        pltpu.VMEM((RPW, IW), I32),
        pltpu.VMEM((NCH, GW), I32),
        pltpu.VMEM((NBUF, GW, H), F32),
        pltpu.VMEM((16,), I32),
    ] + [pltpu.SemaphoreType.DMA] * (2 * NBUF)

    @functools.partial(
        pl.kernel,
        out_type=jax.ShapeDtypeStruct((DISP_ROWS, H), F32),
        mesh=_sc_mesh(),
        scratch_types=scratch,
        compiler_params=pltpu.CompilerParams(needs_layout_passes=False),
    )
    def k(h_hbm, inv_hbm, cnt_hbm, out_hbm, inv_v, idx_v, buf, cnt_v, *sems):
        sin, sout = sems[:NBUF], sems[NBUF:]
        wid = lax.axis_index("s") * 2 + lax.axis_index("c")
        base = wid * RPW
        pltpu.sync_copy(inv_hbm.at[pl.ds(base, RPW)], inv_v)
        pltpu.sync_copy(cnt_hbm, cnt_v)
        e = base // CAP
        ce = plsc.load_gather(cnt_v, [jnp.full((16,), e, I32)])
        bound = e * CAP + jnp.minimum(ce, CAP)
        zeros = jnp.zeros((16,), I32)
        lane = lax.iota(I32, 16)
        for kk in range(RPW // 16):
            s = base + kk * 16 + lane
            ids = plsc.load_gather(inv_v, [kk * 16 + lane, zeros])
            valid = s < bound
            vals = jnp.where(valid, ids, TRASH + (s & (DISP_ROWS - TRASH - 1)))
            idx_v[(kk * 16) // GW, pl.ds((kk * 16) % GW, 16)] = vals

        ins = [None] * NCH
        outs = [None] * NCH
        for j in range(min(NBUF, NCH)):
            ins[j] = pltpu.async_copy(
                h_hbm.at[pl.ds(base + j * GW, GW)], buf.at[j % NBUF],
                sin[j % NBUF])
        for j in range(NCH):
            ins[j].wait()
            outs[j] = pltpu.async_copy(
                buf.at[j % NBUF], out_hbm.at[idx_v.at[j]], sout[j % NBUF])
            nxt = j + NBUF
            if nxt < NCH:
                outs[j].wait()
                ins[nxt] = pltpu.async_copy(
                    h_hbm.at[pl.ds(base + nxt * GW, GW)],
                    buf.at[nxt % NBUF], sin[nxt % NBUF])
        for j in range(max(0, NCH - NBUF), NCH):
            outs[j].wait()

    return k(h, inv16, counts)


def kernel(x, y, W1, b1, Wg1, We1a, be1a, We1b, be1b, W2, b2, Wg2,
           We2a, be2a, We2b, be2b, W3, b3):
    xf = x.reshape(T, H)
    hidden16, ss1, gate1, cnt1, hs = _stage1(xf, W1, b1.reshape(1, H), Wg1)
    disp1, inv1 = _sc_dispatch(hidden16, ss1.reshape(-1, GW))
    h1 = _ffn(disp1, We1a, be1a.reshape(E, 1, H),
              We1b, be1b.reshape(E, 1, H))
    rows1 = _sc_combine(h1, inv1, cnt1.reshape(16))

    out16, ss2, pay, cnt2 = _w2(rows1, gate1, W2, b2.reshape(1, H), Wg2)
    disp2, pay_slot = _sc_dispatch2(out16, ss2.reshape(-1, GW), pay)
    loss = _ffn2_final(disp2, We2a, be2a.reshape(E, 1, H),
                       We2b, be2b.reshape(E, 1, H), pay_slot, cnt2, hs,
                       W3, b3.reshape(1, H), y.reshape(B, 1).astype(I32))
    return loss.reshape(())
